# Initial kernel scaffold; baseline (speedup 1.0000x reference)
import jax
import jax.numpy as jnp
from jax.experimental import pallas as pl


def kernel(x, edge_index, W1_0, W1_1, W1_2, b1, W2_0, W2_1, W2_2, b2):
    raise NotImplementedError("write your pallas kernel here")


# speedup vs baseline: 9.8552x; 9.8552x over previous
import functools

import jax
import jax.numpy as jnp
from jax import lax
from jax.experimental import pallas as pl
from jax.experimental.pallas import tpu as pltpu
from jax.experimental.pallas import tpu_sc as plsc

N = 10000
E = 160000
D = 256
DH = 128
NC = 2
NS = 16
CHUNK = 128
NCHUNK = 79
EP_TILE = NCHUNK * CHUNK
E_PAD = NS * EP_TILE
NP = 10112
ROWS_TILE = NP // NS
DEG_TILE = 5120
E_PAD_DEG = NC * NS * DEG_TILE
DEG_N = NP
STILE = 640
WCW = 80
WCHUNKS = (80, 80, 80, 80, 80, 80, 80, 72)

_f32 = jnp.float32
_mesh = plsc.VectorSubcoreMesh(
    core_axis_name="c", subcore_axis_name="s", num_cores=NC, num_subcores=NS)
_sc_params = pltpu.CompilerParams(needs_layout_passes=False)


def _deg_body(dst_ref, out_ref, dbuf, dacc):
    c = lax.axis_index("c")
    s = lax.axis_index("s")
    wid = c * NS + s
    zeros16 = jnp.zeros((16,), _f32)
    ones16 = jnp.ones((16,), _f32)

    def zbody(j, carry):
        dacc[pl.ds(j * 16, 16)] = zeros16
        return carry
    lax.fori_loop(0, DEG_N // 16, zbody, 0)

    pltpu.sync_copy(dst_ref.at[pl.ds(wid * DEG_TILE, DEG_TILE)], dbuf)

    def ebody(e, carry):
        idx = dbuf[pl.ds(e * 16, 16)]
        plsc.addupdate_scatter(dacc, [idx], ones16)
        return carry
    lax.fori_loop(0, DEG_TILE // 16, ebody, 0)

    pltpu.sync_copy(dacc.at[pl.ds(0, DEG_N)], out_ref.at[wid])


_deg_kernel = pl.kernel(
    _deg_body,
    out_type=[jax.ShapeDtypeStruct((NC * NS, DEG_N), _f32)],
    mesh=_mesh,
    scratch_types=[
        pltpu.VMEM((DEG_TILE,), jnp.int32),
        pltpu.VMEM((DEG_N,), _f32),
    ],
    compiler_params=_sc_params,
)


def _make_prop(cfg):
    n_out = len(cfg)
    n_scale = sum(1 for sc, _ in cfg if sc)
    n_bias = sum(1 for _, b in cfg if b)

    def body(*refs):
        pos = 0
        y_ref, src2_ref, dst_ref = refs[0], refs[1], refs[2]
        pos = 3
        scale_refs = refs[pos:pos + n_scale]; pos += n_scale
        bias_refs = refs[pos:pos + n_bias]; pos += n_bias
        out_refs = refs[pos:pos + n_out]; pos += n_out
        srcbuf, dstbuf, rowbuf, wbuf, sbuf, bbuf, acc, sem = refs[pos:]

        c = lax.axis_index("c")
        s = lax.axis_index("s")
        r0 = s * ROWS_TILE
        crow = c * NP

        pltpu.sync_copy(y_ref.at[pl.ds(crow + r0, ROWS_TILE)],
                        acc.at[pl.ds(r0, ROWS_TILE)])
        plsc.subcore_barrier()

        eoff = s * EP_TILE

        def ebody(k, carry):
            e0 = eoff + k * CHUNK
            pltpu.sync_copy(src2_ref.at[pl.ds(c * E_PAD + e0, CHUNK)], srcbuf)
            pltpu.sync_copy(dst_ref.at[pl.ds(e0, CHUNK)], dstbuf)
            pltpu.async_copy(y_ref.at[srcbuf], rowbuf, sem).wait()
            pltpu.sync_copy(rowbuf, acc.at[dstbuf], add=True)
            return carry
        lax.fori_loop(0, NCHUNK, ebody, 0)
        plsc.subcore_barrier()

        si = 0
        bi = 0
        for oi in range(n_out):
            do_scale, do_bias = cfg[oi]
            dst_slice = out_refs[oi].at[pl.ds(crow + r0, ROWS_TILE)]
            if not do_scale and not do_bias:
                pltpu.sync_copy(acc.at[pl.ds(r0, ROWS_TILE)], dst_slice)
                continue
            if do_scale:
                pltpu.sync_copy(scale_refs[si].at[pl.ds(s * STILE, STILE)],
                                sbuf)
                si += 1
            if do_bias:
                pltpu.sync_copy(bias_refs[bi].at[pl.ds(c * DH, DH)], bbuf)
                bi += 1

            def one_row(r, svb):
                for g in range(DH // 16):
                    v = wbuf[r, pl.ds(g * 16, 16)]
                    if do_scale:
                        v = v * svb
                    if do_bias:
                        v = v + bbuf[pl.ds(g * 16, 16)]
                    wbuf[r, pl.ds(g * 16, 16)] = v

            for k, sz in enumerate(WCHUNKS):
                pltpu.sync_copy(acc.at[pl.ds(r0 + k * WCW, sz)],
                                wbuf.at[pl.ds(0, sz)])

                def gbody(rg, carry, _k=k):
                    if do_scale:
                        sv16 = sbuf[pl.ds(_k * WCW + rg * 8, 16)]
                    for j in range(8):
                        svb = (lax.broadcast(sv16[j], (16,))
                               if do_scale else None)
                        one_row(rg * 8 + j, svb)
                    return carry
                lax.fori_loop(0, sz // 8, gbody, 0)
                pltpu.sync_copy(
                    wbuf.at[pl.ds(0, sz)],
                    out_refs[oi].at[pl.ds(crow + r0 + k * WCW, sz)])

    out_type = [jax.ShapeDtypeStruct((2 * NP, DH), _f32)] * n_out
    scratch = [
        pltpu.VMEM((CHUNK,), jnp.int32),
        pltpu.VMEM((CHUNK,), jnp.int32),
        pltpu.VMEM((CHUNK, DH), _f32),
        pltpu.VMEM((WCW, DH), _f32),
        pltpu.VMEM((STILE,), _f32),
        pltpu.VMEM((DH,), _f32),
        pltpu.VMEM_SHARED((NP, DH), _f32),
        pltpu.SemaphoreType.DMA,
    ]
    return pl.kernel(body, out_type=out_type, mesh=_mesh,
                     scratch_types=scratch, compiler_params=_sc_params)


_prop_plain_scaled = _make_prop(((False, False), (True, False)))
_prop_plain = _make_prop(((False, False),))
_prop_scaled = _make_prop(((True, False),))
_prop_scaled_bias = _make_prop(((True, True),))


_BM = 2000


def _deg_reduce_body(degp_ref, d_ref, d2_ref):
    deg = jnp.sum(degp_ref[...], axis=0, keepdims=True) + 1.0
    dv = lax.rsqrt(deg)
    d_ref[...] = dv
    d2_ref[...] = dv * dv


def _tc_deg_reduce(degp):
    sds = jax.ShapeDtypeStruct((1, DEG_N), _f32)
    return pl.pallas_call(
        _deg_reduce_body,
        out_shape=[sds, sds],
    )(degp)


def _scale_body(x_ref, d_ref, u0_ref):
    u0_ref[...] = x_ref[...] * d_ref[...]


def _tc_scale(x, d):
    g = N // _BM
    return pl.pallas_call(
        _scale_body,
        grid=(g,),
        in_specs=[
            pl.BlockSpec((_BM, D), lambda i: (i, 0)),
            pl.BlockSpec((_BM, 1), lambda i: (i, 0)),
        ],
        out_specs=pl.BlockSpec((_BM, D), lambda i: (i, 0)),
        out_shape=jax.ShapeDtypeStruct((N, D), _f32),
    )(x, d)


def _l1_body(x_ref, t1_ref, t2_ref, d_ref, w0_ref, w1_ref, w2_ref, b_ref,
             h_ref):
    dv = d_ref[...]
    a0 = jnp.dot(x_ref[...], w0_ref[...], preferred_element_type=_f32)
    a1 = jnp.dot(dv * t1_ref[...], w1_ref[...], preferred_element_type=_f32)
    a2 = jnp.dot(dv * t2_ref[...], w2_ref[...], preferred_element_type=_f32)
    h = jnp.concatenate([a0, a1, a2], axis=1) + b_ref[...]
    h_ref[...] = jnp.maximum(h, 0.0)


def _tc_layer1(x, t1, t2, d, w0, w1, w2, b):
    g = N // _BM
    full = lambda r, c: pl.BlockSpec((r, c), lambda i: (0, 0))
    row = lambda c: pl.BlockSpec((_BM, c), lambda i: (i, 0))
    return pl.pallas_call(
        _l1_body,
        grid=(g,),
        in_specs=[row(D), row(D), row(D), row(1),
                  full(D, D), full(D, D), full(D, D), full(1, 3 * D)],
        out_specs=row(3 * D),
        out_shape=jax.ShapeDtypeStruct((N, 3 * D), _f32),
    )(x, t1, t2, d, w0, w1, w2, b)


def _l2_body(h_ref, d_ref, w0_ref, w1_ref, w2_ref, b_ref,
             g0_ref, g1_ref, g2_ref):
    dv = d_ref[...]
    h = h_ref[...]
    g0_ref[...] = jnp.dot(h, w0_ref[...], preferred_element_type=_f32) + b_ref[...]
    g1_ref[...] = dv * jnp.dot(h, w1_ref[...], preferred_element_type=_f32)
    g2_ref[...] = dv * jnp.dot(h, w2_ref[...], preferred_element_type=_f32)


def _tc_layer2(h, d, w0, w1, w2, b):
    g = N // _BM
    full = lambda r, c: pl.BlockSpec((r, c), lambda i: (0, 0))
    row = lambda c: pl.BlockSpec((_BM, c), lambda i: (i, 0))
    sds = jax.ShapeDtypeStruct((N, D), _f32)
    return pl.pallas_call(
        _l2_body,
        grid=(g,),
        in_specs=[row(3 * D), row(1),
                  full(3 * D, D), full(3 * D, D), full(3 * D, D), full(1, D)],
        out_specs=[row(D), row(D), row(D)],
        out_shape=[sds, sds, sds],
    )(h, d, w0, w1, w2, b)


def _cat(a):
    a = jnp.pad(a, ((0, NP - N), (0, 0)))
    return jnp.concatenate([a[:, :DH], a[:, DH:]], axis=0)


def _uncat(a):
    return jnp.concatenate([a[:N], a[NP:NP + N]], axis=1)


def _scale_vec(v):
    v = jnp.pad(v.reshape(-1), (0, NP - N)).reshape(NS, ROWS_TILE)
    return jnp.pad(v, ((0, 0), (0, STILE - ROWS_TILE))).reshape(-1)


def kernel(x, edge_index, W1_0, W1_1, W1_2, b1, W2_0, W2_1, W2_2, b2):
    src = edge_index[0]
    dst = edge_index[1]
    ar = jnp.arange(E_PAD - E, dtype=jnp.int32)
    ar_deg = jnp.arange(E_PAD_DEG - E, dtype=jnp.int32)
    src_p = jnp.concatenate([src, ar % 64])
    dst_p = jnp.concatenate([dst, N + (ar_deg % 16)])
    src2 = jnp.concatenate([src_p, src_p + NP])

    (degp,) = _deg_kernel(dst_p)
    drow, d2row = _tc_deg_reduce(degp)
    d = drow.reshape(DEG_N, 1)[:N]
    d2 = d2row.reshape(DEG_N, 1)[:N]
    u0 = _tc_scale(x, d)
    d_t = _scale_vec(d)
    d2_t = _scale_vec(d2)

    t1c, y2c = _prop_plain_scaled(_cat(u0), src2, dst_p, d2_t)
    (t2c,) = _prop_plain(y2c, src2, dst_p)

    h = _tc_layer1(x, _uncat(t1c), _uncat(t2c), d,
                   W1_0, W1_1, W1_2, b1.reshape(1, 3 * D))
    g0, G1, G2 = _tc_layer2(h, d, W2_0, W2_1, W2_2, b2[:D].reshape(1, D))

    (q1c,) = _prop_scaled_bias(_cat(G1), src2, dst_p, d_t, b2[D:2 * D])
    (y4c,) = _prop_scaled(_cat(G2), src2, dst_p, d2_t)
    (q2c,) = _prop_scaled_bias(y4c, src2, dst_p, d_t, b2[2 * D:])

    return jnp.concatenate([g0, _uncat(q1c), _uncat(q2c)], axis=1)


# speedup vs baseline: 16.1026x; 1.6339x over previous
import functools

import jax
import jax.numpy as jnp
from jax import lax
from jax.experimental import pallas as pl
from jax.experimental.pallas import tpu as pltpu
from jax.experimental.pallas import tpu_sc as plsc

N = 10000
E = 160000
D = 256
DH = 128
NC = 2
NS = 16
CHUNK = 128
NCHUNK = 79
EP_TILE = NCHUNK * CHUNK
E_PAD = NS * EP_TILE
NP = 10112
ROWS_TILE = NP // NS
DEG_TILE = 5120
E_PAD_DEG = NC * NS * DEG_TILE
DEG_N = NP
STILE = 640
WCW = 80
WCHUNKS = (80, 80, 80, 80, 80, 80, 80, 72)

_f32 = jnp.float32
_mesh = plsc.VectorSubcoreMesh(
    core_axis_name="c", subcore_axis_name="s", num_cores=NC, num_subcores=NS)
_sc_params = pltpu.CompilerParams(needs_layout_passes=False)


def _deg_body(dst_ref, out_ref, dbuf, dacc):
    c = lax.axis_index("c")
    s = lax.axis_index("s")
    wid = c * NS + s
    zeros16 = jnp.zeros((16,), _f32)
    ones16 = jnp.ones((16,), _f32)

    def zbody(j, carry):
        dacc[pl.ds(j * 16, 16)] = zeros16
        return carry
    lax.fori_loop(0, DEG_N // 16, zbody, 0)

    pltpu.sync_copy(dst_ref.at[pl.ds(wid * DEG_TILE, DEG_TILE)], dbuf)

    def ebody(e, carry):
        idx = dbuf[pl.ds(e * 16, 16)]
        plsc.addupdate_scatter(dacc, [idx], ones16)
        return carry
    lax.fori_loop(0, DEG_TILE // 16, ebody, 0)

    pltpu.sync_copy(dacc.at[pl.ds(0, DEG_N)], out_ref.at[wid])


_deg_kernel = pl.kernel(
    _deg_body,
    out_type=[jax.ShapeDtypeStruct((NC * NS, DEG_N), _f32)],
    mesh=_mesh,
    scratch_types=[
        pltpu.VMEM((DEG_TILE,), jnp.int32),
        pltpu.VMEM((DEG_N,), _f32),
    ],
    compiler_params=_sc_params,
)


def _make_prop(cfg):
    n_out = len(cfg)
    n_scale = sum(1 for sc, _ in cfg if sc)
    n_bias = sum(1 for _, b in cfg if b)

    def body(*refs):
        pos = 0
        y_ref, src2_ref, dst_ref = refs[0], refs[1], refs[2]
        pos = 3
        scale_refs = refs[pos:pos + n_scale]; pos += n_scale
        bias_refs = refs[pos:pos + n_bias]; pos += n_bias
        out_refs = refs[pos:pos + n_out]; pos += n_out
        (srcbuf, dstbuf, rowbuf, wbuf, sbuf, bbuf, acc,
         isem0, isem1, gsem, ssem0, ssem1) = refs[pos:]
        isems = (isem0, isem1)
        ssems = (ssem0, ssem1)

        c = lax.axis_index("c")
        s = lax.axis_index("s")
        r0 = s * ROWS_TILE
        crow = c * NP

        pltpu.sync_copy(y_ref.at[pl.ds(crow + r0, ROWS_TILE)],
                        acc.at[pl.ds(r0, ROWS_TILE)])
        plsc.subcore_barrier()

        eoff = s * EP_TILE

        idx_h = {}
        scat_h = {}

        def issue_idx(k):
            e0 = eoff + k * CHUNK
            b4 = k % 4
            hs = pltpu.async_copy(
                src2_ref.at[pl.ds(c * E_PAD + e0, CHUNK)],
                srcbuf.at[b4], isems[k % 2])
            hd = pltpu.async_copy(
                dst_ref.at[pl.ds(e0, CHUNK)], dstbuf.at[b4], isems[k % 2])
            idx_h[k] = (hs, hd)

        issue_idx(0)
        issue_idx(1)
        for k in range(NCHUNK):
            b2 = k % 2
            b4 = k % 4
            for h in idx_h.pop(k):
                h.wait()
            if k >= 2:
                scat_h.pop(k - 2).wait()
            if k + 2 < NCHUNK:
                issue_idx(k + 2)
            pltpu.async_copy(y_ref.at[srcbuf.at[b4]],
                             rowbuf.at[b2], gsem).wait()
            scat_h[k] = pltpu.async_copy(
                rowbuf.at[b2], acc.at[dstbuf.at[b4]], ssems[b2], add=True)
        for k in sorted(scat_h):
            scat_h.pop(k).wait()
        plsc.subcore_barrier()

        si = 0
        bi = 0
        for oi in range(n_out):
            do_scale, do_bias = cfg[oi]
            dst_slice = out_refs[oi].at[pl.ds(crow + r0, ROWS_TILE)]
            if not do_scale and not do_bias:
                pltpu.sync_copy(acc.at[pl.ds(r0, ROWS_TILE)], dst_slice)
                continue
            if do_scale:
                pltpu.sync_copy(scale_refs[si].at[pl.ds(s * STILE, STILE)],
                                sbuf)
                si += 1
            if do_bias:
                pltpu.sync_copy(bias_refs[bi].at[pl.ds(c * DH, DH)], bbuf)
                bi += 1

            def one_row(r, svb):
                for g in range(DH // 16):
                    v = wbuf[r, pl.ds(g * 16, 16)]
                    if do_scale:
                        v = v * svb
                    if do_bias:
                        v = v + bbuf[pl.ds(g * 16, 16)]
                    wbuf[r, pl.ds(g * 16, 16)] = v

            for k, sz in enumerate(WCHUNKS):
                pltpu.sync_copy(acc.at[pl.ds(r0 + k * WCW, sz)],
                                wbuf.at[pl.ds(0, sz)])

                def gbody(rg, carry, _k=k):
                    if do_scale:
                        sv16 = sbuf[pl.ds(_k * WCW + rg * 8, 16)]
                    for j in range(8):
                        svb = (lax.broadcast(sv16[j], (16,))
                               if do_scale else None)
                        one_row(rg * 8 + j, svb)
                    return carry
                lax.fori_loop(0, sz // 8, gbody, 0)
                pltpu.sync_copy(
                    wbuf.at[pl.ds(0, sz)],
                    out_refs[oi].at[pl.ds(crow + r0 + k * WCW, sz)])

    out_type = [jax.ShapeDtypeStruct((2 * NP, DH), _f32)] * n_out
    scratch = [
        pltpu.VMEM((4, CHUNK), jnp.int32),
        pltpu.VMEM((4, CHUNK), jnp.int32),
        pltpu.VMEM((2, CHUNK, DH), _f32),
        pltpu.VMEM((WCW, DH), _f32),
        pltpu.VMEM((STILE,), _f32),
        pltpu.VMEM((DH,), _f32),
        pltpu.VMEM_SHARED((NP, DH), _f32),
        pltpu.SemaphoreType.DMA,
        pltpu.SemaphoreType.DMA,
        pltpu.SemaphoreType.DMA,
        pltpu.SemaphoreType.DMA,
        pltpu.SemaphoreType.DMA,
    ]
    return pl.kernel(body, out_type=out_type, mesh=_mesh,
                     scratch_types=scratch, compiler_params=_sc_params)


_prop_plain_scaled = _make_prop(((False, False), (True, False)))
_prop_plain = _make_prop(((False, False),))
_prop_scaled = _make_prop(((True, False),))
_prop_scaled_bias = _make_prop(((True, True),))


_BM = 2000


def _deg_reduce_body(degp_ref, d_ref, d2_ref):
    deg = jnp.sum(degp_ref[...], axis=0, keepdims=True) + 1.0
    dv = lax.rsqrt(deg)
    d_ref[...] = dv
    d2_ref[...] = dv * dv


def _tc_deg_reduce(degp):
    sds = jax.ShapeDtypeStruct((1, DEG_N), _f32)
    return pl.pallas_call(
        _deg_reduce_body,
        out_shape=[sds, sds],
    )(degp)


def _scale_body(x_ref, d_ref, u0_ref):
    u0_ref[...] = x_ref[...] * d_ref[...]


def _tc_scale(x, d):
    g = N // _BM
    return pl.pallas_call(
        _scale_body,
        grid=(g,),
        in_specs=[
            pl.BlockSpec((_BM, D), lambda i: (i, 0)),
            pl.BlockSpec((_BM, 1), lambda i: (i, 0)),
        ],
        out_specs=pl.BlockSpec((_BM, D), lambda i: (i, 0)),
        out_shape=jax.ShapeDtypeStruct((N, D), _f32),
    )(x, d)


def _l1_body(x_ref, t1_ref, t2_ref, d_ref, w0_ref, w1_ref, w2_ref, b_ref,
             h_ref):
    dv = d_ref[...]
    a0 = jnp.dot(x_ref[...], w0_ref[...], preferred_element_type=_f32)
    a1 = jnp.dot(dv * t1_ref[...], w1_ref[...], preferred_element_type=_f32)
    a2 = jnp.dot(dv * t2_ref[...], w2_ref[...], preferred_element_type=_f32)
    h = jnp.concatenate([a0, a1, a2], axis=1) + b_ref[...]
    h_ref[...] = jnp.maximum(h, 0.0)


def _tc_layer1(x, t1, t2, d, w0, w1, w2, b):
    g = N // _BM
    full = lambda r, c: pl.BlockSpec((r, c), lambda i: (0, 0))
    row = lambda c: pl.BlockSpec((_BM, c), lambda i: (i, 0))
    return pl.pallas_call(
        _l1_body,
        grid=(g,),
        in_specs=[row(D), row(D), row(D), row(1),
                  full(D, D), full(D, D), full(D, D), full(1, 3 * D)],
        out_specs=row(3 * D),
        out_shape=jax.ShapeDtypeStruct((N, 3 * D), _f32),
    )(x, t1, t2, d, w0, w1, w2, b)


def _l2_body(h_ref, d_ref, w0_ref, w1_ref, w2_ref, b_ref,
             g0_ref, g1_ref, g2_ref):
    dv = d_ref[...]
    h = h_ref[...]
    g0_ref[...] = jnp.dot(h, w0_ref[...], preferred_element_type=_f32) + b_ref[...]
    g1_ref[...] = dv * jnp.dot(h, w1_ref[...], preferred_element_type=_f32)
    g2_ref[...] = dv * jnp.dot(h, w2_ref[...], preferred_element_type=_f32)


def _tc_layer2(h, d, w0, w1, w2, b):
    g = N // _BM
    full = lambda r, c: pl.BlockSpec((r, c), lambda i: (0, 0))
    row = lambda c: pl.BlockSpec((_BM, c), lambda i: (i, 0))
    sds = jax.ShapeDtypeStruct((N, D), _f32)
    return pl.pallas_call(
        _l2_body,
        grid=(g,),
        in_specs=[row(3 * D), row(1),
                  full(3 * D, D), full(3 * D, D), full(3 * D, D), full(1, D)],
        out_specs=[row(D), row(D), row(D)],
        out_shape=[sds, sds, sds],
    )(h, d, w0, w1, w2, b)


def _cat(a):
    a = jnp.pad(a, ((0, NP - N), (0, 0)))
    return jnp.concatenate([a[:, :DH], a[:, DH:]], axis=0)


def _uncat(a):
    return jnp.concatenate([a[:N], a[NP:NP + N]], axis=1)


def _scale_vec(v):
    v = jnp.pad(v.reshape(-1), (0, NP - N)).reshape(NS, ROWS_TILE)
    return jnp.pad(v, ((0, 0), (0, STILE - ROWS_TILE))).reshape(-1)


def kernel(x, edge_index, W1_0, W1_1, W1_2, b1, W2_0, W2_1, W2_2, b2):
    src = edge_index[0]
    dst = edge_index[1]
    ar = jnp.arange(E_PAD - E, dtype=jnp.int32)
    ar_deg = jnp.arange(E_PAD_DEG - E, dtype=jnp.int32)
    src_p = jnp.concatenate([src, ar % 64])
    dst_p = jnp.concatenate([dst, N + (ar_deg % 16)])
    src2 = jnp.concatenate([src_p, src_p + NP])

    (degp,) = _deg_kernel(dst_p)
    drow, d2row = _tc_deg_reduce(degp)
    d = drow.reshape(DEG_N, 1)[:N]
    d2 = d2row.reshape(DEG_N, 1)[:N]
    u0 = _tc_scale(x, d)
    d_t = _scale_vec(d)
    d2_t = _scale_vec(d2)

    t1c, y2c = _prop_plain_scaled(_cat(u0), src2, dst_p, d2_t)
    (t2c,) = _prop_plain(y2c, src2, dst_p)

    h = _tc_layer1(x, _uncat(t1c), _uncat(t2c), d,
                   W1_0, W1_1, W1_2, b1.reshape(1, 3 * D))
    g0, G1, G2 = _tc_layer2(h, d, W2_0, W2_1, W2_2, b2[:D].reshape(1, D))

    (q1c,) = _prop_scaled_bias(_cat(G1), src2, dst_p, d_t, b2[D:2 * D])
    (y4c,) = _prop_scaled(_cat(G2), src2, dst_p, d2_t)
    (q2c,) = _prop_scaled_bias(y4c, src2, dst_p, d_t, b2[2 * D:])

    return jnp.concatenate([g0, _uncat(q1c), _uncat(q2c)], axis=1)


# speedup vs baseline: 16.3282x; 1.0140x over previous
import functools

import jax
import jax.numpy as jnp
from jax import lax
from jax.experimental import pallas as pl
from jax.experimental.pallas import tpu as pltpu
from jax.experimental.pallas import tpu_sc as plsc

N = 10000
E = 160000
D = 256
DH = 128
NC = 2
NS = 16
CHUNK = 128
NCHUNK = 79
EP_TILE = NCHUNK * CHUNK
E_PAD = NS * EP_TILE
NP = 10112
ROWS_TILE = NP // NS
DEG_TILE = 5120
E_PAD_DEG = NC * NS * DEG_TILE
DEG_N = NP
STILE = 640
WCW = 48
WCHUNKS = (48,) * 13 + (8,)

_f32 = jnp.float32
_mesh = plsc.VectorSubcoreMesh(
    core_axis_name="c", subcore_axis_name="s", num_cores=NC, num_subcores=NS)
_sc_params = pltpu.CompilerParams(needs_layout_passes=False)


def _deg_body(dst_ref, out_ref, dbuf, dacc):
    c = lax.axis_index("c")
    s = lax.axis_index("s")
    wid = c * NS + s
    zeros16 = jnp.zeros((16,), _f32)
    ones16 = jnp.ones((16,), _f32)

    def zbody(j, carry):
        dacc[pl.ds(j * 16, 16)] = zeros16
        return carry
    lax.fori_loop(0, DEG_N // 16, zbody, 0)

    pltpu.sync_copy(dst_ref.at[pl.ds(wid * DEG_TILE, DEG_TILE)], dbuf)

    def ebody(e, carry):
        idx = dbuf[pl.ds(e * 16, 16)]
        plsc.addupdate_scatter(dacc, [idx], ones16)
        return carry
    lax.fori_loop(0, DEG_TILE // 16, ebody, 0)

    pltpu.sync_copy(dacc.at[pl.ds(0, DEG_N)], out_ref.at[wid])


_deg_kernel = pl.kernel(
    _deg_body,
    out_type=[jax.ShapeDtypeStruct((NC * NS, DEG_N), _f32)],
    mesh=_mesh,
    scratch_types=[
        pltpu.VMEM((DEG_TILE,), jnp.int32),
        pltpu.VMEM((DEG_N,), _f32),
    ],
    compiler_params=_sc_params,
)


def _make_prop(cfg):
    n_out = len(cfg)
    n_scale = sum(1 for sc, _ in cfg if sc)
    n_bias = sum(1 for _, b in cfg if b)

    def body(*refs):
        pos = 0
        y_ref, src2_ref, dst_ref = refs[0], refs[1], refs[2]
        pos = 3
        scale_refs = refs[pos:pos + n_scale]; pos += n_scale
        bias_refs = refs[pos:pos + n_bias]; pos += n_bias
        out_refs = refs[pos:pos + n_out]; pos += n_out
        (srcbuf, dstbuf, rowbuf, wbuf, sbuf, bbuf, acc,
         isem0, isem1, gsem, ssem0, ssem1, wisem, wosem0, wosem1) = refs[pos:]
        isems = (isem0, isem1)
        ssems = (ssem0, ssem1)
        wosems = (wosem0, wosem1)

        c = lax.axis_index("c")
        s = lax.axis_index("s")
        r0 = s * ROWS_TILE
        crow = c * NP

        pltpu.sync_copy(y_ref.at[pl.ds(crow + r0, ROWS_TILE)],
                        acc.at[pl.ds(r0, ROWS_TILE)])
        plsc.subcore_barrier()

        eoff = s * EP_TILE

        idx_h = {}
        scat_h = {}

        def issue_idx(k):
            e0 = eoff + k * CHUNK
            b4 = k % 4
            hs = pltpu.async_copy(
                src2_ref.at[pl.ds(c * E_PAD + e0, CHUNK)],
                srcbuf.at[b4], isems[k % 2])
            hd = pltpu.async_copy(
                dst_ref.at[pl.ds(e0, CHUNK)], dstbuf.at[b4], isems[k % 2])
            idx_h[k] = (hs, hd)

        issue_idx(0)
        issue_idx(1)
        for k in range(NCHUNK):
            b2 = k % 2
            b4 = k % 4
            for h in idx_h.pop(k):
                h.wait()
            if k >= 2:
                scat_h.pop(k - 2).wait()
            if k + 2 < NCHUNK:
                issue_idx(k + 2)
            pltpu.async_copy(y_ref.at[srcbuf.at[b4]],
                             rowbuf.at[b2], gsem).wait()
            scat_h[k] = pltpu.async_copy(
                rowbuf.at[b2], acc.at[dstbuf.at[b4]], ssems[b2], add=True)
        for k in sorted(scat_h):
            scat_h.pop(k).wait()
        plsc.subcore_barrier()

        si = 0
        bi = 0
        for oi in range(n_out):
            do_scale, do_bias = cfg[oi]
            dst_slice = out_refs[oi].at[pl.ds(crow + r0, ROWS_TILE)]
            if not do_scale and not do_bias:
                pltpu.sync_copy(acc.at[pl.ds(r0, ROWS_TILE)], dst_slice)
                continue
            if do_scale:
                pltpu.sync_copy(scale_refs[si].at[pl.ds(s * STILE, STILE)],
                                sbuf)
                si += 1
            if do_bias:
                pltpu.sync_copy(bias_refs[bi].at[pl.ds(c * DH, DH)], bbuf)
                bi += 1

            def one_row(wb, r, svb):
                for g in range(DH // 16):
                    v = wb[r, pl.ds(g * 16, 16)]
                    if do_scale:
                        v = v * svb
                    if do_bias:
                        v = v + bbuf[pl.ds(g * 16, 16)]
                    wb[r, pl.ds(g * 16, 16)] = v

            out_h = {}
            for k, sz in enumerate(WCHUNKS):
                b = k % 2
                if k >= 2:
                    out_h.pop(k - 2).wait()
                wb = wbuf.at[b]
                pltpu.async_copy(acc.at[pl.ds(r0 + k * WCW, sz)],
                                 wb.at[pl.ds(0, sz)], wisem).wait()

                def gbody(rg, carry, _k=k, _wb=wb):
                    if do_scale:
                        sv16 = sbuf[pl.ds(_k * WCW + rg * 8, 16)]
                    for j in range(8):
                        svb = (lax.broadcast(sv16[j], (16,))
                               if do_scale else None)
                        one_row(_wb, rg * 8 + j, svb)
                    return carry
                lax.fori_loop(0, sz // 8, gbody, 0)
                out_h[k] = pltpu.async_copy(
                    wb.at[pl.ds(0, sz)],
                    out_refs[oi].at[pl.ds(crow + r0 + k * WCW, sz)],
                    wosems[b])
            for k in sorted(out_h):
                out_h.pop(k).wait()

    out_type = [jax.ShapeDtypeStruct((2 * NP, DH), _f32)] * n_out
    scratch = [
        pltpu.VMEM((4, CHUNK), jnp.int32),
        pltpu.VMEM((4, CHUNK), jnp.int32),
        pltpu.VMEM((2, CHUNK, DH), _f32),
        pltpu.VMEM((2, WCW, DH), _f32),
        pltpu.VMEM((STILE,), _f32),
        pltpu.VMEM((DH,), _f32),
        pltpu.VMEM_SHARED((NP, DH), _f32),
    ] + [pltpu.SemaphoreType.DMA] * 8
    return pl.kernel(body, out_type=out_type, mesh=_mesh,
                     scratch_types=scratch, compiler_params=_sc_params)


_prop_plain_scaled = _make_prop(((False, False), (True, False)))
_prop_plain = _make_prop(((False, False),))
_prop_scaled = _make_prop(((True, False),))
_prop_scaled_bias = _make_prop(((True, True),))


_BMN = 1264
_GB = NP // _BMN


def _deg_reduce_body(degp_ref, d_ref, d2_ref):
    deg = jnp.sum(degp_ref[...], axis=0, keepdims=True) + 1.0
    dv = lax.rsqrt(deg)
    d_ref[...] = dv
    d2_ref[...] = dv * dv


def _tc_deg_reduce(degp):
    sds = jax.ShapeDtypeStruct((1, DEG_N), _f32)
    return pl.pallas_call(
        _deg_reduce_body,
        out_shape=[sds, sds],
    )(degp)


def _scale_body(x_ref, d_ref, u0_ref):
    u0_ref[...] = x_ref[...] * d_ref[...]


def _tc_scale(x_p, d_np):
    return pl.pallas_call(
        _scale_body,
        grid=(_GB, 2),
        in_specs=[
            pl.BlockSpec((_BMN, DH), lambda i, j: (i, j)),
            pl.BlockSpec((_BMN, 1), lambda i, j: (i, 0)),
        ],
        out_specs=pl.BlockSpec((_BMN, DH), lambda i, j: (j * _GB + i, 0)),
        out_shape=jax.ShapeDtypeStruct((2 * NP, DH), _f32),
    )(x_p, d_np)


def _l1_body(x_ref, t1l_ref, t1r_ref, t2l_ref, t2r_ref, d_ref,
             w0_ref, w1_ref, w2_ref, b_ref, h_ref):
    dv = d_ref[...]
    w1 = w1_ref[...]
    w2 = w2_ref[...]
    a0 = jnp.dot(x_ref[...], w0_ref[...], preferred_element_type=_f32)
    a1 = (jnp.dot(dv * t1l_ref[...], w1[:DH], preferred_element_type=_f32)
          + jnp.dot(dv * t1r_ref[...], w1[DH:], preferred_element_type=_f32))
    a2 = (jnp.dot(dv * t2l_ref[...], w2[:DH], preferred_element_type=_f32)
          + jnp.dot(dv * t2r_ref[...], w2[DH:], preferred_element_type=_f32))
    h = jnp.concatenate([a0, a1, a2], axis=1) + b_ref[...]
    h_ref[...] = jnp.maximum(h, 0.0)


def _tc_layer1(x_p, t1c, t2c, d_np, w0, w1, w2, b):
    full = lambda r, c: pl.BlockSpec((r, c), lambda i: (0, 0))
    left = pl.BlockSpec((_BMN, DH), lambda i: (i, 0))
    right = pl.BlockSpec((_BMN, DH), lambda i: (i + _GB, 0))
    return pl.pallas_call(
        _l1_body,
        grid=(_GB,),
        in_specs=[pl.BlockSpec((_BMN, D), lambda i: (i, 0)),
                  left, right, left, right,
                  pl.BlockSpec((_BMN, 1), lambda i: (i, 0)),
                  full(D, D), full(D, D), full(D, D), full(1, 3 * D)],
        out_specs=pl.BlockSpec((_BMN, 3 * D), lambda i: (i, 0)),
        out_shape=jax.ShapeDtypeStruct((NP, 3 * D), _f32),
    )(x_p, t1c, t1c, t2c, t2c, d_np, w0, w1, w2, b)


def _l2_body(h_ref, d_ref, w0_ref, w1_ref, w2_ref, b_ref,
             g0_ref, g1_ref, g2_ref):
    dv = d_ref[...]
    h = h_ref[...]
    g0_ref[...] = (jnp.dot(h, w0_ref[...], preferred_element_type=_f32)
                   + b_ref[...])
    g1_ref[...] = dv * jnp.dot(h, w1_ref[...], preferred_element_type=_f32)
    g2_ref[...] = dv * jnp.dot(h, w2_ref[...], preferred_element_type=_f32)


def _tc_layer2(h_p, d_np, w0, w1, w2, b):
    wspec = pl.BlockSpec((3 * D, DH), lambda i, j: (0, j))
    catspec = pl.BlockSpec((_BMN, DH), lambda i, j: (j * _GB + i, 0))
    return pl.pallas_call(
        _l2_body,
        grid=(_GB, 2),
        in_specs=[pl.BlockSpec((_BMN, 3 * D), lambda i, j: (i, 0)),
                  pl.BlockSpec((_BMN, 1), lambda i, j: (i, 0)),
                  wspec, wspec, wspec,
                  pl.BlockSpec((1, DH), lambda i, j: (0, j))],
        out_specs=[pl.BlockSpec((_BMN, DH), lambda i, j: (i, j)),
                   catspec, catspec],
        out_shape=[jax.ShapeDtypeStruct((NP, D), _f32),
                   jax.ShapeDtypeStruct((2 * NP, DH), _f32),
                   jax.ShapeDtypeStruct((2 * NP, DH), _f32)],
    )(h_p, d_np, w0, w1, w2, b)


def _scale_vec(v):
    v = v.reshape(NS, ROWS_TILE)
    return jnp.pad(v, ((0, 0), (0, STILE - ROWS_TILE))).reshape(-1)


def kernel(x, edge_index, W1_0, W1_1, W1_2, b1, W2_0, W2_1, W2_2, b2):
    src = edge_index[0]
    dst = edge_index[1]
    ar = jnp.arange(E_PAD - E, dtype=jnp.int32)
    ar_deg = jnp.arange(E_PAD_DEG - E, dtype=jnp.int32)
    src_p = jnp.concatenate([src, ar % 64])
    dst_p = jnp.concatenate([dst, N + (ar_deg % 16)])
    src2 = jnp.concatenate([src_p, src_p + NP])

    (degp,) = _deg_kernel(dst_p)
    drow, d2row = _tc_deg_reduce(degp)
    d_np = drow.reshape(NP, 1)
    d_t = _scale_vec(drow)
    d2_t = _scale_vec(d2row)

    x_p = jnp.pad(x, ((0, NP - N), (0, 0)))
    u0c = _tc_scale(x_p, d_np)

    t1c, y2c = _prop_plain_scaled(u0c, src2, dst_p, d2_t)
    (t2c,) = _prop_plain(y2c, src2, dst_p)

    h_p = _tc_layer1(x_p, t1c, t2c, d_np,
                     W1_0, W1_1, W1_2, b1.reshape(1, 3 * D))
    g0_p, G1c, G2c = _tc_layer2(h_p, d_np, W2_0, W2_1, W2_2,
                                b2[:D].reshape(1, D))

    (q1c,) = _prop_scaled_bias(G1c, src2, dst_p, d_t, b2[D:2 * D])
    (y4c,) = _prop_scaled(G2c, src2, dst_p, d2_t)
    (q2c,) = _prop_scaled_bias(y4c, src2, dst_p, d_t, b2[2 * D:])

    return jnp.concatenate([g0_p[:N], q1c[:N], q1c[NP:NP + N],
                            q2c[:N], q2c[NP:NP + N]], axis=1)


# speedup vs baseline: 18.6737x; 1.1436x over previous
import functools

import jax
import jax.numpy as jnp
from jax import lax
from jax.experimental import pallas as pl
from jax.experimental.pallas import tpu as pltpu
from jax.experimental.pallas import tpu_sc as plsc

N = 10000
E = 160000
D = 256
DH = 128
NC = 2
NS = 16
CHUNK = 128
NCHUNK = 79
EP_TILE = NCHUNK * CHUNK
E_PAD = NS * EP_TILE
NCB = E_PAD // CHUNK
NP = 10112
ROWS_TILE = NP // NS
DEG_TILE = 5120
E_PAD_DEG = NC * NS * DEG_TILE
DEG_N = NP
STILE = 640
WCW = 48
WCHUNKS = (48,) * 13 + (8,)

_f32 = jnp.float32
_mesh = plsc.VectorSubcoreMesh(
    core_axis_name="c", subcore_axis_name="s", num_cores=NC, num_subcores=NS)
_sc_params = pltpu.CompilerParams(needs_layout_passes=False)


def _deg_body(dst_ref, out_ref, dbuf, dacc):
    c = lax.axis_index("c")
    s = lax.axis_index("s")
    wid = c * NS + s
    zeros16 = jnp.zeros((16,), _f32)
    ones16 = jnp.ones((16,), _f32)

    def zbody(j, carry):
        dacc[pl.ds(j * 16, 16)] = zeros16
        return carry
    lax.fori_loop(0, DEG_N // 16, zbody, 0)

    pltpu.sync_copy(dst_ref.at[pl.ds(wid * DEG_TILE, DEG_TILE)], dbuf)

    def ebody(e, carry):
        idx = dbuf[pl.ds(e * 16, 16)]
        plsc.addupdate_scatter(dacc, [idx], ones16)
        return carry
    lax.fori_loop(0, DEG_TILE // 16, ebody, 0)

    pltpu.sync_copy(dacc.at[pl.ds(0, DEG_N)], out_ref.at[wid])


_deg_kernel = pl.kernel(
    _deg_body,
    out_type=[jax.ShapeDtypeStruct((NC * NS, DEG_N), _f32)],
    mesh=_mesh,
    scratch_types=[
        pltpu.VMEM((DEG_TILE,), jnp.int32),
        pltpu.VMEM((DEG_N,), _f32),
    ],
    compiler_params=_sc_params,
)


def _make_prop(cfg):
    n_out = len(cfg)
    n_scale = sum(1 for sc, _ in cfg if sc)
    n_bias = sum(1 for _, b in cfg if b)

    def body(*refs):
        pos = 0
        y_ref, epack_ref = refs[0], refs[1]
        pos = 2
        scale_refs = refs[pos:pos + n_scale]; pos += n_scale
        bias_refs = refs[pos:pos + n_bias]; pos += n_bias
        out_refs = refs[pos:pos + n_out]; pos += n_out
        (ebuf, rowbuf, wbuf, sbuf, bbuf, acc,
         isem0, isem1, isem2, gsem0, gsem1, ssem0, ssem1,
         wisem, wosem0, wosem1) = refs[pos:]
        isems = (isem0, isem1, isem2)
        gsems = (gsem0, gsem1)
        ssems = (ssem0, ssem1)
        wosems = (wosem0, wosem1)

        c = lax.axis_index("c")
        s = lax.axis_index("s")
        r0 = s * ROWS_TILE
        crow = c * NP

        pltpu.sync_copy(y_ref.at[pl.ds(crow + r0, ROWS_TILE)],
                        acc.at[pl.ds(r0, ROWS_TILE)])
        plsc.subcore_barrier()

        idx_h = {}
        gat_h = {}
        scat_h = {}
        kchunk0 = s * NCHUNK

        def issue_idx(k):
            idx_h[k] = pltpu.async_copy(
                epack_ref.at[c * NCB + kchunk0 + k], ebuf.at[k % 4],
                isems[k % 3])

        def issue_gather(k):
            gat_h[k] = pltpu.async_copy(
                y_ref.at[ebuf.at[k % 4, 0]], rowbuf.at[k % 2],
                gsems[k % 2])

        def issue_scat(k):
            scat_h[k] = pltpu.async_copy(
                rowbuf.at[k % 2], acc.at[ebuf.at[k % 4, 1]],
                ssems[k % 2], add=True)

        issue_idx(0)
        issue_idx(1)
        issue_idx(2)
        idx_h.pop(0).wait()
        issue_gather(0)
        for k in range(1, NCHUNK):
            idx_h.pop(k).wait()
            if k >= 2:
                scat_h.pop(k - 2).wait()
            issue_gather(k)
            if k + 2 < NCHUNK:
                issue_idx(k + 2)
            gat_h.pop(k - 1).wait()
            issue_scat(k - 1)
        gat_h.pop(NCHUNK - 1).wait()
        issue_scat(NCHUNK - 1)
        for k in sorted(scat_h):
            scat_h.pop(k).wait()
        plsc.subcore_barrier()

        si = 0
        bi = 0
        for oi in range(n_out):
            do_scale, do_bias = cfg[oi]
            dst_slice = out_refs[oi].at[pl.ds(crow + r0, ROWS_TILE)]
            if not do_scale and not do_bias:
                pltpu.sync_copy(acc.at[pl.ds(r0, ROWS_TILE)], dst_slice)
                continue
            if do_scale:
                pltpu.sync_copy(scale_refs[si].at[pl.ds(s * STILE, STILE)],
                                sbuf)
                si += 1
            if do_bias:
                pltpu.sync_copy(bias_refs[bi].at[pl.ds(c * DH, DH)], bbuf)
                bi += 1

            def one_row(wb, r, svb):
                for g in range(DH // 16):
                    v = wb[r, pl.ds(g * 16, 16)]
                    if do_scale:
                        v = v * svb
                    if do_bias:
                        v = v + bbuf[pl.ds(g * 16, 16)]
                    wb[r, pl.ds(g * 16, 16)] = v

            out_h = {}
            for k, sz in enumerate(WCHUNKS):
                b = k % 2
                if k >= 2:
                    out_h.pop(k - 2).wait()
                wb = wbuf.at[b]
                pltpu.async_copy(acc.at[pl.ds(r0 + k * WCW, sz)],
                                 wb.at[pl.ds(0, sz)], wisem).wait()

                def gbody(rg, carry, _k=k, _wb=wb):
                    if do_scale:
                        sv16 = sbuf[pl.ds(_k * WCW + rg * 8, 16)]
                    for j in range(8):
                        svb = (lax.broadcast(sv16[j], (16,))
                               if do_scale else None)
                        one_row(_wb, rg * 8 + j, svb)
                    return carry
                lax.fori_loop(0, sz // 8, gbody, 0)
                out_h[k] = pltpu.async_copy(
                    wb.at[pl.ds(0, sz)],
                    out_refs[oi].at[pl.ds(crow + r0 + k * WCW, sz)],
                    wosems[b])
            for k in sorted(out_h):
                out_h.pop(k).wait()

    out_type = [jax.ShapeDtypeStruct((2 * NP, DH), _f32)] * n_out
    scratch = [
        pltpu.VMEM((4, 2, CHUNK), jnp.int32),
        pltpu.VMEM((2, CHUNK, DH), _f32),
        pltpu.VMEM((2, WCW, DH), _f32),
        pltpu.VMEM((STILE,), _f32),
        pltpu.VMEM((DH,), _f32),
        pltpu.VMEM_SHARED((NP, DH), _f32),
    ] + [pltpu.SemaphoreType.DMA] * 10
    return pl.kernel(body, out_type=out_type, mesh=_mesh,
                     scratch_types=scratch, compiler_params=_sc_params)


_prop_plain_scaled = _make_prop(((False, False), (True, False)))
_prop_plain = _make_prop(((False, False),))
_prop_scaled = _make_prop(((True, False),))
_prop_scaled_bias = _make_prop(((True, True),))


_BMN = 1264
_GB = NP // _BMN


def _deg_reduce_body(degp_ref, d_ref, d2_ref):
    deg = jnp.sum(degp_ref[...], axis=0, keepdims=True) + 1.0
    dv = lax.rsqrt(deg)
    d_ref[...] = dv
    d2_ref[...] = dv * dv


def _tc_deg_reduce(degp):
    sds = jax.ShapeDtypeStruct((1, DEG_N), _f32)
    return pl.pallas_call(
        _deg_reduce_body,
        out_shape=[sds, sds],
    )(degp)


def _scale_body(x_ref, d_ref, u0_ref):
    u0_ref[...] = x_ref[...] * d_ref[...]


def _tc_scale(x_p, d_np):
    return pl.pallas_call(
        _scale_body,
        grid=(_GB, 2),
        in_specs=[
            pl.BlockSpec((_BMN, DH), lambda i, j: (i, j)),
            pl.BlockSpec((_BMN, 1), lambda i, j: (i, 0)),
        ],
        out_specs=pl.BlockSpec((_BMN, DH), lambda i, j: (j * _GB + i, 0)),
        out_shape=jax.ShapeDtypeStruct((2 * NP, DH), _f32),
    )(x_p, d_np)


def _l1_body(x_ref, t1l_ref, t1r_ref, t2l_ref, t2r_ref, d_ref,
             w0_ref, w1_ref, w2_ref, b_ref, h_ref):
    dv = d_ref[...]
    w1 = w1_ref[...]
    w2 = w2_ref[...]
    a0 = jnp.dot(x_ref[...], w0_ref[...], preferred_element_type=_f32)
    a1 = (jnp.dot(dv * t1l_ref[...], w1[:DH], preferred_element_type=_f32)
          + jnp.dot(dv * t1r_ref[...], w1[DH:], preferred_element_type=_f32))
    a2 = (jnp.dot(dv * t2l_ref[...], w2[:DH], preferred_element_type=_f32)
          + jnp.dot(dv * t2r_ref[...], w2[DH:], preferred_element_type=_f32))
    h = jnp.concatenate([a0, a1, a2], axis=1) + b_ref[...]
    h_ref[...] = jnp.maximum(h, 0.0)


def _tc_layer1(x_p, t1c, t2c, d_np, w0, w1, w2, b):
    full = lambda r, c: pl.BlockSpec((r, c), lambda i: (0, 0))
    left = pl.BlockSpec((_BMN, DH), lambda i: (i, 0))
    right = pl.BlockSpec((_BMN, DH), lambda i: (i + _GB, 0))
    return pl.pallas_call(
        _l1_body,
        grid=(_GB,),
        in_specs=[pl.BlockSpec((_BMN, D), lambda i: (i, 0)),
                  left, right, left, right,
                  pl.BlockSpec((_BMN, 1), lambda i: (i, 0)),
                  full(D, D), full(D, D), full(D, D), full(1, 3 * D)],
        out_specs=pl.BlockSpec((_BMN, 3 * D), lambda i: (i, 0)),
        out_shape=jax.ShapeDtypeStruct((NP, 3 * D), _f32),
    )(x_p, t1c, t1c, t2c, t2c, d_np, w0, w1, w2, b)


def _l2_body(h_ref, d_ref, w0_ref, w1_ref, w2_ref, b_ref,
             g0_ref, g1_ref, g2_ref):
    dv = d_ref[...]
    h = h_ref[...]
    g0_ref[...] = (jnp.dot(h, w0_ref[...], preferred_element_type=_f32)
                   + b_ref[...])
    g1_ref[...] = dv * jnp.dot(h, w1_ref[...], preferred_element_type=_f32)
    g2_ref[...] = dv * jnp.dot(h, w2_ref[...], preferred_element_type=_f32)


def _tc_layer2(h_p, d_np, w0, w1, w2, b):
    wspec = pl.BlockSpec((3 * D, DH), lambda i, j: (0, j))
    catspec = pl.BlockSpec((_BMN, DH), lambda i, j: (j * _GB + i, 0))
    return pl.pallas_call(
        _l2_body,
        grid=(_GB, 2),
        in_specs=[pl.BlockSpec((_BMN, 3 * D), lambda i, j: (i, 0)),
                  pl.BlockSpec((_BMN, 1), lambda i, j: (i, 0)),
                  wspec, wspec, wspec,
                  pl.BlockSpec((1, DH), lambda i, j: (0, j))],
        out_specs=[pl.BlockSpec((_BMN, DH), lambda i, j: (i, j)),
                   catspec, catspec],
        out_shape=[jax.ShapeDtypeStruct((NP, D), _f32),
                   jax.ShapeDtypeStruct((2 * NP, DH), _f32),
                   jax.ShapeDtypeStruct((2 * NP, DH), _f32)],
    )(h_p, d_np, w0, w1, w2, b)


def _scale_vec(v):
    v = v.reshape(NS, ROWS_TILE)
    return jnp.pad(v, ((0, 0), (0, STILE - ROWS_TILE))).reshape(-1)


def kernel(x, edge_index, W1_0, W1_1, W1_2, b1, W2_0, W2_1, W2_2, b2):
    src = edge_index[0]
    dst = edge_index[1]
    ar = jnp.arange(E_PAD - E, dtype=jnp.int32)
    ar_deg = jnp.arange(E_PAD_DEG - E, dtype=jnp.int32)
    src_p = jnp.concatenate([src, ar % 64])
    dst_p = jnp.concatenate([dst, N + (ar_deg % 16)])
    src2 = jnp.concatenate([src_p, src_p + NP]).reshape(2 * NCB, CHUNK)
    dst2 = jnp.concatenate([dst_p[:E_PAD]] * 2).reshape(2 * NCB, CHUNK)
    epack = jnp.stack([src2, dst2], axis=1)

    (degp,) = _deg_kernel(dst_p)
    drow, d2row = _tc_deg_reduce(degp)
    d_np = drow.reshape(NP, 1)
    d_t = _scale_vec(drow)
    d2_t = _scale_vec(d2row)

    x_p = jnp.pad(x, ((0, NP - N), (0, 0)))
    u0c = _tc_scale(x_p, d_np)

    t1c, y2c = _prop_plain_scaled(u0c, epack, d2_t)
    (t2c,) = _prop_plain(y2c, epack)

    h_p = _tc_layer1(x_p, t1c, t2c, d_np,
                     W1_0, W1_1, W1_2, b1.reshape(1, 3 * D))
    g0_p, G1c, G2c = _tc_layer2(h_p, d_np, W2_0, W2_1, W2_2,
                                b2[:D].reshape(1, D))

    (q1c,) = _prop_scaled_bias(G1c, epack, d_t, b2[D:2 * D])
    (y4c,) = _prop_scaled(G2c, epack, d2_t)
    (q2c,) = _prop_scaled_bias(y4c, epack, d_t, b2[2 * D:])

    return jnp.concatenate([g0_p[:N], q1c[:N], q1c[NP:NP + N],
                            q2c[:N], q2c[NP:NP + N]], axis=1)


# speedup vs baseline: 18.6949x; 1.0011x over previous
import functools

import jax
import jax.numpy as jnp
from jax import lax
from jax.experimental import pallas as pl
from jax.experimental.pallas import tpu as pltpu
from jax.experimental.pallas import tpu_sc as plsc

N = 10000
E = 160000
D = 256
DH = 128
NC = 2
NS = 16
CHUNK = 128
NCHUNK = 79
EP_TILE = NCHUNK * CHUNK
E_PAD = NS * EP_TILE
NCB = E_PAD // CHUNK
NP = 10112
ROWS_TILE = NP // NS
DEG_TILE = 5120
E_PAD_DEG = NC * NS * DEG_TILE
DEG_N = NP
STILE = 640
WCW = 48
WCHUNKS = (48,) * 13 + (8,)

_f32 = jnp.float32
_mesh = plsc.VectorSubcoreMesh(
    core_axis_name="c", subcore_axis_name="s", num_cores=NC, num_subcores=NS)
_sc_params = pltpu.CompilerParams(needs_layout_passes=False)


def _deg_body(dst_ref, out_ref, dbuf, dacc):
    c = lax.axis_index("c")
    s = lax.axis_index("s")
    wid = c * NS + s
    zeros16 = jnp.zeros((16,), _f32)
    ones16 = jnp.ones((16,), _f32)

    def zbody(j, carry):
        dacc[pl.ds(j * 16, 16)] = zeros16
        return carry
    lax.fori_loop(0, DEG_N // 16, zbody, 0)

    pltpu.sync_copy(dst_ref.at[pl.ds(wid * DEG_TILE, DEG_TILE)], dbuf)

    def ebody(e, carry):
        idx = dbuf[pl.ds(e * 16, 16)]
        plsc.addupdate_scatter(dacc, [idx], ones16)
        return carry
    lax.fori_loop(0, DEG_TILE // 16, ebody, 0)

    pltpu.sync_copy(dacc.at[pl.ds(0, DEG_N)], out_ref.at[wid])


_deg_kernel = pl.kernel(
    _deg_body,
    out_type=[jax.ShapeDtypeStruct((NC * NS, DEG_N), _f32)],
    mesh=_mesh,
    scratch_types=[
        pltpu.VMEM((DEG_TILE,), jnp.int32),
        pltpu.VMEM((DEG_N,), _f32),
    ],
    compiler_params=_sc_params,
)


def _make_prop(rounds, n_tables):
    all_cfg = [c for _, outs in rounds for c in outs]
    n_out = len(all_cfg)
    n_scale = sum(1 for sc, _ in all_cfg if sc)
    n_bias = sum(1 for _, b in all_cfg if b)

    def body(*refs):
        pos = 0
        tables = refs[pos:pos + n_tables]; pos += n_tables
        epack_ref = refs[pos]; pos += 1
        scale_refs = refs[pos:pos + n_scale]; pos += n_scale
        bias_refs = refs[pos:pos + n_bias]; pos += n_bias
        out_refs = refs[pos:pos + n_out]; pos += n_out
        (ebuf, rowbuf, wbuf, sbuf, bbuf, acc,
         isem0, isem1, isem2, gsem0, gsem1, ssem0, ssem1,
         wisem, wosem0, wosem1) = refs[pos:]
        isems = (isem0, isem1, isem2)
        gsems = (gsem0, gsem1)
        ssems = (ssem0, ssem1)
        wosems = (wosem0, wosem1)

        c = lax.axis_index("c")
        s = lax.axis_index("s")
        r0 = s * ROWS_TILE
        crow = c * NP
        kchunk0 = s * NCHUNK
        si = 0
        bi = 0
        oi = 0

        for tspec, outs_cfg in rounds:
            y_ref = (tables[tspec[1]] if tspec[0] == "in"
                     else out_refs[tspec[1]])

            pltpu.sync_copy(y_ref.at[pl.ds(crow + r0, ROWS_TILE)],
                            acc.at[pl.ds(r0, ROWS_TILE)])
            plsc.subcore_barrier()

            idx_h = {}
            gat_h = {}
            scat_h = {}

            def issue_idx(k):
                idx_h[k] = pltpu.async_copy(
                    epack_ref.at[c * NCB + kchunk0 + k], ebuf.at[k % 4],
                    isems[k % 3])

            def issue_gather(k):
                gat_h[k] = pltpu.async_copy(
                    y_ref.at[ebuf.at[k % 4, 0]], rowbuf.at[k % 2],
                    gsems[k % 2])

            def issue_scat(k):
                scat_h[k] = pltpu.async_copy(
                    rowbuf.at[k % 2], acc.at[ebuf.at[k % 4, 1]],
                    ssems[k % 2], add=True)

            issue_idx(0)
            issue_idx(1)
            issue_idx(2)
            idx_h.pop(0).wait()
            issue_gather(0)
            for k in range(1, NCHUNK):
                idx_h.pop(k).wait()
                if k >= 2:
                    scat_h.pop(k - 2).wait()
                issue_gather(k)
                if k + 2 < NCHUNK:
                    issue_idx(k + 2)
                gat_h.pop(k - 1).wait()
                issue_scat(k - 1)
            gat_h.pop(NCHUNK - 1).wait()
            issue_scat(NCHUNK - 1)
            for k in sorted(scat_h):
                scat_h.pop(k).wait()
            plsc.subcore_barrier()

            for do_scale, do_bias in outs_cfg:
                dst_full = out_refs[oi].at[pl.ds(crow + r0, ROWS_TILE)]
                if not do_scale and not do_bias:
                    pltpu.sync_copy(acc.at[pl.ds(r0, ROWS_TILE)], dst_full)
                    oi += 1
                    continue
                if do_scale:
                    pltpu.sync_copy(
                        scale_refs[si].at[pl.ds(s * STILE, STILE)], sbuf)
                    si += 1
                if do_bias:
                    pltpu.sync_copy(bias_refs[bi].at[pl.ds(c * DH, DH)],
                                    bbuf)
                    bi += 1

                def one_row(wb, r, svb, _sc=do_scale, _bs=do_bias):
                    for g in range(DH // 16):
                        v = wb[r, pl.ds(g * 16, 16)]
                        if _sc:
                            v = v * svb
                        if _bs:
                            v = v + bbuf[pl.ds(g * 16, 16)]
                        wb[r, pl.ds(g * 16, 16)] = v

                out_h = {}
                for k, sz in enumerate(WCHUNKS):
                    b = k % 2
                    if k >= 2:
                        out_h.pop(k - 2).wait()
                    wb = wbuf.at[b]
                    pltpu.async_copy(acc.at[pl.ds(r0 + k * WCW, sz)],
                                     wb.at[pl.ds(0, sz)], wisem).wait()

                    def gbody(rg, carry, _k=k, _wb=wb, _sc=do_scale):
                        if _sc:
                            sv16 = sbuf[pl.ds(_k * WCW + rg * 8, 16)]
                        for j in range(8):
                            svb = (lax.broadcast(sv16[j], (16,))
                                   if _sc else None)
                            one_row(_wb, rg * 8 + j, svb)
                        return carry
                    lax.fori_loop(0, sz // 8, gbody, 0)
                    out_h[k] = pltpu.async_copy(
                        wb.at[pl.ds(0, sz)],
                        out_refs[oi].at[pl.ds(crow + r0 + k * WCW, sz)],
                        wosems[b])
                for k in sorted(out_h):
                    out_h.pop(k).wait()
                oi += 1

    out_type = [jax.ShapeDtypeStruct((2 * NP, DH), _f32)] * n_out
    scratch = [
        pltpu.VMEM((4, 2, CHUNK), jnp.int32),
        pltpu.VMEM((2, CHUNK, DH), _f32),
        pltpu.VMEM((2, WCW, DH), _f32),
        pltpu.VMEM((STILE,), _f32),
        pltpu.VMEM((DH,), _f32),
        pltpu.VMEM_SHARED((NP, DH), _f32),
    ] + [pltpu.SemaphoreType.DMA] * 10
    return pl.kernel(body, out_type=out_type, mesh=_mesh,
                     scratch_types=scratch, compiler_params=_sc_params)


_prop_plain_scaled = _make_prop(((("in", 0), ((False, False), (True, False))),),
                                n_tables=1)
_prop_plain = _make_prop(((("in", 0), ((False, False),)),), n_tables=1)
_prop_scaled = _make_prop(((("in", 0), ((True, False),)),), n_tables=1)
_prop_scaled_bias = _make_prop(((("in", 0), ((True, True),)),), n_tables=1)


_BMN = 1264
_GB = NP // _BMN


def _deg_reduce_body(degp_ref, d_ref, d2_ref):
    deg = jnp.sum(degp_ref[...], axis=0, keepdims=True) + 1.0
    dv = lax.rsqrt(deg)
    d_ref[...] = dv
    d2_ref[...] = dv * dv


def _tc_deg_reduce(degp):
    sds = jax.ShapeDtypeStruct((1, DEG_N), _f32)
    return pl.pallas_call(
        _deg_reduce_body,
        out_shape=[sds, sds],
    )(degp)


def _scale_body(x_ref, d_ref, u0_ref):
    u0_ref[...] = x_ref[...] * d_ref[...]


def _tc_scale(x_p, d_np):
    return pl.pallas_call(
        _scale_body,
        grid=(_GB, 2),
        in_specs=[
            pl.BlockSpec((_BMN, DH), lambda i, j: (i, j)),
            pl.BlockSpec((_BMN, 1), lambda i, j: (i, 0)),
        ],
        out_specs=pl.BlockSpec((_BMN, DH), lambda i, j: (j * _GB + i, 0)),
        out_shape=jax.ShapeDtypeStruct((2 * NP, DH), _f32),
    )(x_p, d_np)


_bf16 = jnp.bfloat16


def _bdot(a, b):
    return jnp.dot(a.astype(_bf16), b.astype(_bf16),
                   preferred_element_type=_f32)


def _l1_body(x_ref, t1l_ref, t1r_ref, t2l_ref, t2r_ref, d_ref,
             w0_ref, w1_ref, w2_ref, b_ref, h_ref):
    dv = d_ref[...]
    w1 = w1_ref[...]
    w2 = w2_ref[...]
    a0 = _bdot(x_ref[...], w0_ref[...])
    a1 = _bdot(dv * t1l_ref[...], w1[:DH]) + _bdot(dv * t1r_ref[...], w1[DH:])
    a2 = _bdot(dv * t2l_ref[...], w2[:DH]) + _bdot(dv * t2r_ref[...], w2[DH:])
    h = jnp.concatenate([a0, a1, a2], axis=1) + b_ref[...]
    h_ref[...] = jnp.maximum(h, 0.0)


def _tc_layer1(x_p, t1c, t2c, d_np, w0, w1, w2, b):
    full = lambda r, c: pl.BlockSpec((r, c), lambda i: (0, 0))
    left = pl.BlockSpec((_BMN, DH), lambda i: (i, 0))
    right = pl.BlockSpec((_BMN, DH), lambda i: (i + _GB, 0))
    return pl.pallas_call(
        _l1_body,
        grid=(_GB,),
        in_specs=[pl.BlockSpec((_BMN, D), lambda i: (i, 0)),
                  left, right, left, right,
                  pl.BlockSpec((_BMN, 1), lambda i: (i, 0)),
                  full(D, D), full(D, D), full(D, D), full(1, 3 * D)],
        out_specs=pl.BlockSpec((_BMN, 3 * D), lambda i: (i, 0)),
        out_shape=jax.ShapeDtypeStruct((NP, 3 * D), _f32),
    )(x_p, t1c, t1c, t2c, t2c, d_np, w0, w1, w2, b)


def _l2_body(h_ref, d_ref, w0_ref, w1_ref, w2_ref, b_ref,
             g0_ref, g1_ref, g2_ref):
    dv = d_ref[...]
    h = h_ref[...].astype(_bf16)
    g0_ref[...] = (jnp.dot(h, w0_ref[...].astype(_bf16),
                           preferred_element_type=_f32) + b_ref[...])
    g1_ref[...] = dv * jnp.dot(h, w1_ref[...].astype(_bf16),
                               preferred_element_type=_f32)
    g2_ref[...] = dv * jnp.dot(h, w2_ref[...].astype(_bf16),
                               preferred_element_type=_f32)


def _tc_layer2(h_p, d_np, w0, w1, w2, b):
    wspec = pl.BlockSpec((3 * D, DH), lambda i, j: (0, j))
    catspec = pl.BlockSpec((_BMN, DH), lambda i, j: (j * _GB + i, 0))
    return pl.pallas_call(
        _l2_body,
        grid=(_GB, 2),
        in_specs=[pl.BlockSpec((_BMN, 3 * D), lambda i, j: (i, 0)),
                  pl.BlockSpec((_BMN, 1), lambda i, j: (i, 0)),
                  wspec, wspec, wspec,
                  pl.BlockSpec((1, DH), lambda i, j: (0, j))],
        out_specs=[pl.BlockSpec((_BMN, DH), lambda i, j: (i, j)),
                   catspec, catspec],
        out_shape=[jax.ShapeDtypeStruct((NP, D), _f32),
                   jax.ShapeDtypeStruct((2 * NP, DH), _f32),
                   jax.ShapeDtypeStruct((2 * NP, DH), _f32)],
    )(h_p, d_np, w0, w1, w2, b)


def _scale_vec(v):
    v = v.reshape(NS, ROWS_TILE)
    return jnp.pad(v, ((0, 0), (0, STILE - ROWS_TILE))).reshape(-1)


def kernel(x, edge_index, W1_0, W1_1, W1_2, b1, W2_0, W2_1, W2_2, b2):
    src = edge_index[0]
    dst = edge_index[1]
    ar = jnp.arange(E_PAD - E, dtype=jnp.int32)
    ar_deg = jnp.arange(E_PAD_DEG - E, dtype=jnp.int32)
    src_p = jnp.concatenate([src, ar % 64])
    dst_p = jnp.concatenate([dst, N + (ar_deg % 16)])
    src2 = jnp.concatenate([src_p, src_p + NP]).reshape(2 * NCB, CHUNK)
    dst2 = jnp.concatenate([dst_p[:E_PAD]] * 2).reshape(2 * NCB, CHUNK)
    epack = jnp.stack([src2, dst2], axis=1)

    (degp,) = _deg_kernel(dst_p)
    drow, d2row = _tc_deg_reduce(degp)
    d_np = drow.reshape(NP, 1)
    d_t = _scale_vec(drow)
    d2_t = _scale_vec(d2row)

    x_p = jnp.pad(x, ((0, NP - N), (0, 0)))
    u0c = _tc_scale(x_p, d_np)

    t1c, y2c = _prop_plain_scaled(u0c, epack, d2_t)
    (t2c,) = _prop_plain(y2c, epack)

    h_p = _tc_layer1(x_p, t1c, t2c, d_np,
                     W1_0, W1_1, W1_2, b1.reshape(1, 3 * D))
    g0_p, G1c, G2c = _tc_layer2(h_p, d_np, W2_0, W2_1, W2_2,
                                b2[:D].reshape(1, D))

    (q1c,) = _prop_scaled_bias(G1c, epack, d_t, b2[D:2 * D])
    (y4c,) = _prop_scaled(G2c, epack, d2_t)
    (q2c,) = _prop_scaled_bias(y4c, epack, d_t, b2[2 * D:])

    return jnp.concatenate([g0_p[:N], q1c[:N], q1c[NP:NP + N],
                            q2c[:N], q2c[NP:NP + N]], axis=1)


# speedup vs baseline: 18.8218x; 1.0068x over previous
import functools

import jax
import jax.numpy as jnp
from jax import lax
from jax.experimental import pallas as pl
from jax.experimental.pallas import tpu as pltpu
from jax.experimental.pallas import tpu_sc as plsc

N = 10000
E = 160000
D = 256
DH = 128
NC = 2
NS = 16
CHUNK = 128
NCHUNK = 79
EP_TILE = NCHUNK * CHUNK
E_PAD = NS * EP_TILE
NCB = E_PAD // CHUNK
NP = 10112
ROWS_TILE = NP // NS
DEG_TILE = 5120
E_PAD_DEG = NC * NS * DEG_TILE
DEG_N = NP
STILE = 640
WCW = 48
WCHUNKS = (48,) * 13 + (8,)

_f32 = jnp.float32
_mesh = plsc.VectorSubcoreMesh(
    core_axis_name="c", subcore_axis_name="s", num_cores=NC, num_subcores=NS)
_sc_params = pltpu.CompilerParams(needs_layout_passes=False)


def _deg_body(dst_ref, out_ref, dbuf, dacc):
    c = lax.axis_index("c")
    s = lax.axis_index("s")
    wid = c * NS + s
    zeros16 = jnp.zeros((16,), _f32)
    ones16 = jnp.ones((16,), _f32)

    def zbody(j, carry):
        dacc[pl.ds(j * 16, 16)] = zeros16
        return carry
    lax.fori_loop(0, DEG_N // 16, zbody, 0)

    pltpu.sync_copy(dst_ref.at[pl.ds(wid * DEG_TILE, DEG_TILE)], dbuf)

    def ebody(e, carry):
        idx = dbuf[pl.ds(e * 16, 16)]
        plsc.addupdate_scatter(dacc, [idx], ones16)
        return carry
    lax.fori_loop(0, DEG_TILE // 16, ebody, 0)

    pltpu.sync_copy(dacc.at[pl.ds(0, DEG_N)], out_ref.at[wid])


_deg_kernel = pl.kernel(
    _deg_body,
    out_type=[jax.ShapeDtypeStruct((NC * NS, DEG_N), _f32)],
    mesh=_mesh,
    scratch_types=[
        pltpu.VMEM((DEG_TILE,), jnp.int32),
        pltpu.VMEM((DEG_N,), _f32),
    ],
    compiler_params=_sc_params,
)


def _make_prop(rounds, n_tables):
    all_cfg = [c for _, outs in rounds for c in outs]
    n_out = len(all_cfg)
    n_scale = sum(1 for sc, _ in all_cfg if sc)
    n_bias = sum(1 for _, b in all_cfg if b)

    def body(*refs):
        pos = 0
        tables = refs[pos:pos + n_tables]; pos += n_tables
        epack_ref = refs[pos]; pos += 1
        scale_refs = refs[pos:pos + n_scale]; pos += n_scale
        bias_refs = refs[pos:pos + n_bias]; pos += n_bias
        out_refs = refs[pos:pos + n_out]; pos += n_out
        (ebuf, rowbuf, wbuf, sbuf, bbuf, acc,
         isem0, isem1, isem2, gsem0, gsem1, ssem0, ssem1,
         wisem, wosem0, wosem1) = refs[pos:]
        isems = (isem0, isem1, isem2)
        gsems = (gsem0, gsem1)
        ssems = (ssem0, ssem1)
        wosems = (wosem0, wosem1)

        c = lax.axis_index("c")
        s = lax.axis_index("s")
        r0 = s * ROWS_TILE
        crow = c * NP
        kchunk0 = s * NCHUNK
        si = 0
        bi = 0
        oi = 0

        for tspec, outs_cfg in rounds:
            y_ref = (tables[tspec[1]] if tspec[0] == "in"
                     else out_refs[tspec[1]])

            pltpu.sync_copy(y_ref.at[pl.ds(crow + r0, ROWS_TILE)],
                            acc.at[pl.ds(r0, ROWS_TILE)])
            plsc.subcore_barrier()

            idx_h = {}
            gat_h = {}
            scat_h = {}

            def issue_idx(k):
                idx_h[k] = pltpu.async_copy(
                    epack_ref.at[c * NCB + kchunk0 + k], ebuf.at[k % 4],
                    isems[k % 3])

            def issue_gather(k):
                gat_h[k] = pltpu.async_copy(
                    y_ref.at[ebuf.at[k % 4, 0]], rowbuf.at[k % 2],
                    gsems[k % 2])

            def issue_scat(k):
                scat_h[k] = pltpu.async_copy(
                    rowbuf.at[k % 2], acc.at[ebuf.at[k % 4, 1]],
                    ssems[k % 2], add=True)

            issue_idx(0)
            issue_idx(1)
            issue_idx(2)
            idx_h.pop(0).wait()
            issue_gather(0)
            for k in range(1, NCHUNK):
                idx_h.pop(k).wait()
                if k >= 2:
                    scat_h.pop(k - 2).wait()
                issue_gather(k)
                if k + 2 < NCHUNK:
                    issue_idx(k + 2)
                gat_h.pop(k - 1).wait()
                issue_scat(k - 1)
            gat_h.pop(NCHUNK - 1).wait()
            issue_scat(NCHUNK - 1)
            for k in sorted(scat_h):
                scat_h.pop(k).wait()
            plsc.subcore_barrier()

            for do_scale, do_bias in outs_cfg:
                dst_full = out_refs[oi].at[pl.ds(crow + r0, ROWS_TILE)]
                if not do_scale and not do_bias:
                    pltpu.sync_copy(acc.at[pl.ds(r0, ROWS_TILE)], dst_full)
                    oi += 1
                    continue
                if do_scale:
                    pltpu.sync_copy(
                        scale_refs[si].at[pl.ds(s * STILE, STILE)], sbuf)
                    si += 1
                if do_bias:
                    pltpu.sync_copy(bias_refs[bi].at[pl.ds(c * DH, DH)],
                                    bbuf)
                    bi += 1

                def one_row(wb, r, svb, _sc=do_scale, _bs=do_bias):
                    for g in range(DH // 16):
                        v = wb[r, pl.ds(g * 16, 16)]
                        if _sc:
                            v = v * svb
                        if _bs:
                            v = v + bbuf[pl.ds(g * 16, 16)]
                        wb[r, pl.ds(g * 16, 16)] = v

                out_h = {}
                for k, sz in enumerate(WCHUNKS):
                    b = k % 2
                    if k >= 2:
                        out_h.pop(k - 2).wait()
                    wb = wbuf.at[b]
                    pltpu.async_copy(acc.at[pl.ds(r0 + k * WCW, sz)],
                                     wb.at[pl.ds(0, sz)], wisem).wait()

                    def gbody(rg, carry, _k=k, _wb=wb, _sc=do_scale):
                        if _sc:
                            sv16 = sbuf[pl.ds(_k * WCW + rg * 8, 16)]
                        for j in range(8):
                            svb = (lax.broadcast(sv16[j], (16,))
                                   if _sc else None)
                            one_row(_wb, rg * 8 + j, svb)
                        return carry
                    lax.fori_loop(0, sz // 8, gbody, 0)
                    out_h[k] = pltpu.async_copy(
                        wb.at[pl.ds(0, sz)],
                        out_refs[oi].at[pl.ds(crow + r0 + k * WCW, sz)],
                        wosems[b])
                for k in sorted(out_h):
                    out_h.pop(k).wait()
                oi += 1

    out_type = [jax.ShapeDtypeStruct((2 * NP, DH), _f32)] * n_out
    scratch = [
        pltpu.VMEM((4, 2, CHUNK), jnp.int32),
        pltpu.VMEM((2, CHUNK, DH), _f32),
        pltpu.VMEM((2, WCW, DH), _f32),
        pltpu.VMEM((STILE,), _f32),
        pltpu.VMEM((DH,), _f32),
        pltpu.VMEM_SHARED((NP, DH), _f32),
    ] + [pltpu.SemaphoreType.DMA] * 10
    return pl.kernel(body, out_type=out_type, mesh=_mesh,
                     scratch_types=scratch, compiler_params=_sc_params)


_prop_plain_scaled = _make_prop(((("in", 0), ((False, False), (True, False))),),
                                n_tables=1)
_prop_plain = _make_prop(((("in", 0), ((False, False),)),), n_tables=1)
_prop_scaled = _make_prop(((("in", 0), ((True, False),)),), n_tables=1)
_prop_scaled_bias = _make_prop(((("in", 0), ((True, True),)),), n_tables=1)


_BMN = 1264
_GB = NP // _BMN


def _deg_reduce_body(degp_ref, d_ref, d2_ref):
    deg = jnp.sum(degp_ref[...], axis=0, keepdims=True) + 1.0
    dv = lax.rsqrt(deg)
    d_ref[...] = dv
    d2_ref[...] = dv * dv


def _tc_deg_reduce(degp):
    sds = jax.ShapeDtypeStruct((1, DEG_N), _f32)
    return pl.pallas_call(
        _deg_reduce_body,
        out_shape=[sds, sds],
    )(degp)


def _scale_body(x_ref, d_ref, u0_ref):
    u0_ref[...] = x_ref[...] * d_ref[...]


def _tc_scale(x_p, d_np):
    return pl.pallas_call(
        _scale_body,
        grid=(_GB, 2),
        in_specs=[
            pl.BlockSpec((_BMN, DH), lambda i, j: (i, j)),
            pl.BlockSpec((_BMN, 1), lambda i, j: (i, 0)),
        ],
        out_specs=pl.BlockSpec((_BMN, DH), lambda i, j: (j * _GB + i, 0)),
        out_shape=jax.ShapeDtypeStruct((2 * NP, DH), _f32),
    )(x_p, d_np)


_bf16 = jnp.bfloat16


def _bdot(a, b):
    return jnp.dot(a.astype(_bf16), b.astype(_bf16),
                   preferred_element_type=_f32)


def _l1_body(x_ref, t1l_ref, t1r_ref, t2l_ref, t2r_ref, d_ref,
             w0_ref, w1_ref, w2_ref, b_ref, h_ref):
    dv = d_ref[...]
    w1 = w1_ref[...]
    w2 = w2_ref[...]
    a0 = _bdot(x_ref[...], w0_ref[...])
    a1 = _bdot(dv * t1l_ref[...], w1[:DH]) + _bdot(dv * t1r_ref[...], w1[DH:])
    a2 = _bdot(dv * t2l_ref[...], w2[:DH]) + _bdot(dv * t2r_ref[...], w2[DH:])
    h = jnp.concatenate([a0, a1, a2], axis=1) + b_ref[...]
    h_ref[...] = jnp.maximum(h, 0.0).astype(_bf16)


def _tc_layer1(x_p, t1c, t2c, d_np, w0, w1, w2, b):
    full = lambda r, c: pl.BlockSpec((r, c), lambda i: (0, 0))
    left = pl.BlockSpec((_BMN, DH), lambda i: (i, 0))
    right = pl.BlockSpec((_BMN, DH), lambda i: (i + _GB, 0))
    return pl.pallas_call(
        _l1_body,
        grid=(_GB,),
        in_specs=[pl.BlockSpec((_BMN, D), lambda i: (i, 0)),
                  left, right, left, right,
                  pl.BlockSpec((_BMN, 1), lambda i: (i, 0)),
                  full(D, D), full(D, D), full(D, D), full(1, 3 * D)],
        out_specs=pl.BlockSpec((_BMN, 3 * D), lambda i: (i, 0)),
        out_shape=jax.ShapeDtypeStruct((NP, 3 * D), _bf16),
    )(x_p, t1c, t1c, t2c, t2c, d_np, w0, w1, w2, b)


def _l2_body(h_ref, d_ref, w0_ref, w1_ref, w2_ref, b_ref,
             g0_ref, g1_ref, g2_ref):
    dv = d_ref[...]
    h = h_ref[...]
    g0_ref[...] = (jnp.dot(h, w0_ref[...].astype(_bf16),
                           preferred_element_type=_f32) + b_ref[...])
    g1_ref[...] = dv * jnp.dot(h, w1_ref[...].astype(_bf16),
                               preferred_element_type=_f32)
    g2_ref[...] = dv * jnp.dot(h, w2_ref[...].astype(_bf16),
                               preferred_element_type=_f32)


def _tc_layer2(h_p, d_np, w0, w1, w2, b):
    wspec = pl.BlockSpec((3 * D, DH), lambda i, j: (0, j))
    catspec = pl.BlockSpec((_BMN, DH), lambda i, j: (j * _GB + i, 0))
    return pl.pallas_call(
        _l2_body,
        grid=(_GB, 2),
        in_specs=[pl.BlockSpec((_BMN, 3 * D), lambda i, j: (i, 0)),
                  pl.BlockSpec((_BMN, 1), lambda i, j: (i, 0)),
                  wspec, wspec, wspec,
                  pl.BlockSpec((1, DH), lambda i, j: (0, j))],
        out_specs=[pl.BlockSpec((_BMN, DH), lambda i, j: (i, j)),
                   catspec, catspec],
        out_shape=[jax.ShapeDtypeStruct((NP, D), _f32),
                   jax.ShapeDtypeStruct((2 * NP, DH), _f32),
                   jax.ShapeDtypeStruct((2 * NP, DH), _f32)],
    )(h_p, d_np, w0, w1, w2, b)


def _scale_vec(v):
    v = v.reshape(NS, ROWS_TILE)
    return jnp.pad(v, ((0, 0), (0, STILE - ROWS_TILE))).reshape(-1)


def kernel(x, edge_index, W1_0, W1_1, W1_2, b1, W2_0, W2_1, W2_2, b2):
    src = edge_index[0]
    dst = edge_index[1]
    ar = jnp.arange(E_PAD - E, dtype=jnp.int32)
    ar_deg = jnp.arange(E_PAD_DEG - E, dtype=jnp.int32)
    src_p = jnp.concatenate([src, ar % 64])
    dst_p = jnp.concatenate([dst, N + (ar_deg % 16)])
    src2 = jnp.concatenate([src_p, src_p + NP]).reshape(2 * NCB, CHUNK)
    dst2 = jnp.concatenate([dst_p[:E_PAD]] * 2).reshape(2 * NCB, CHUNK)
    epack = jnp.stack([src2, dst2], axis=1)

    (degp,) = _deg_kernel(dst_p)
    drow, d2row = _tc_deg_reduce(degp)
    d_np = drow.reshape(NP, 1)
    d_t = _scale_vec(drow)
    d2_t = _scale_vec(d2row)

    x_p = jnp.pad(x, ((0, NP - N), (0, 0)))
    u0c = _tc_scale(x_p, d_np)

    t1c, y2c = _prop_plain_scaled(u0c, epack, d2_t)
    (t2c,) = _prop_plain(y2c, epack)

    h_p = _tc_layer1(x_p, t1c, t2c, d_np,
                     W1_0, W1_1, W1_2, b1.reshape(1, 3 * D))
    g0_p, G1c, G2c = _tc_layer2(h_p, d_np, W2_0, W2_1, W2_2,
                                b2[:D].reshape(1, D))

    (q1c,) = _prop_scaled_bias(G1c, epack, d_t, b2[D:2 * D])
    (y4c,) = _prop_scaled(G2c, epack, d2_t)
    (q2c,) = _prop_scaled_bias(y4c, epack, d_t, b2[2 * D:])

    return jnp.concatenate([g0_p[:N], q1c[:N], q1c[NP:NP + N],
                            q2c[:N], q2c[NP:NP + N]], axis=1)


# speedup vs baseline: 19.0619x; 1.0128x over previous
import functools

import jax
import jax.numpy as jnp
from jax import lax
from jax.experimental import pallas as pl
from jax.experimental.pallas import tpu as pltpu
from jax.experimental.pallas import tpu_sc as plsc

N = 10000
E = 160000
D = 256
DH = 128
NC = 2
NS = 16
CHUNK = 128
NCHUNK = 79
EP_TILE = NCHUNK * CHUNK
E_PAD = NS * EP_TILE
NCB = E_PAD // CHUNK
NP = 10112
ROWS_TILE = NP // NS
DEG_TILE = 5120
E_PAD_DEG = NC * NS * DEG_TILE
DEG_N = NP
STILE = 640
WCW = 48
WCHUNKS = (48,) * 13 + (8,)

_f32 = jnp.float32
_mesh = plsc.VectorSubcoreMesh(
    core_axis_name="c", subcore_axis_name="s", num_cores=NC, num_subcores=NS)
_sc_params = pltpu.CompilerParams(needs_layout_passes=False)


def _deg_body(dst_ref, out_ref, dbuf, dacc):
    c = lax.axis_index("c")
    s = lax.axis_index("s")
    wid = c * NS + s
    zeros16 = jnp.zeros((16,), _f32)
    ones16 = jnp.ones((16,), _f32)

    def zbody(j, carry):
        dacc[pl.ds(j * 16, 16)] = zeros16
        return carry
    lax.fori_loop(0, DEG_N // 16, zbody, 0)

    pltpu.sync_copy(dst_ref.at[pl.ds(wid * DEG_TILE, DEG_TILE)], dbuf)

    def ebody(e, carry):
        idx = dbuf[pl.ds(e * 16, 16)]
        plsc.addupdate_scatter(dacc, [idx], ones16)
        return carry
    lax.fori_loop(0, DEG_TILE // 16, ebody, 0)

    pltpu.sync_copy(dacc.at[pl.ds(0, DEG_N)], out_ref.at[wid])


_deg_kernel = pl.kernel(
    _deg_body,
    out_type=[jax.ShapeDtypeStruct((NC * NS, DEG_N), _f32)],
    mesh=_mesh,
    scratch_types=[
        pltpu.VMEM((DEG_TILE,), jnp.int32),
        pltpu.VMEM((DEG_N,), _f32),
    ],
    compiler_params=_sc_params,
)


def _make_prop(rounds, n_tables):
    all_cfg = [c for _, outs in rounds for c in outs]
    n_out = len(all_cfg)
    n_scale = sum(1 for sc, _ in all_cfg if sc)
    n_bias = sum(1 for _, b in all_cfg if b)

    def body(*refs):
        pos = 0
        tables = refs[pos:pos + n_tables]; pos += n_tables
        epack_ref = refs[pos]; pos += 1
        scale_refs = refs[pos:pos + n_scale]; pos += n_scale
        bias_refs = refs[pos:pos + n_bias]; pos += n_bias
        out_refs = refs[pos:pos + n_out]; pos += n_out
        (ebuf, rowbuf, wbuf, sbuf, bbuf, acc,
         isem0, isem1, isem2, gsem0, gsem1, ssem0, ssem1,
         wisem, wosem0, wosem1) = refs[pos:]
        isems = (isem0, isem1, isem2)
        gsems = (gsem0, gsem1)
        ssems = (ssem0, ssem1)
        wosems = (wosem0, wosem1)

        c = lax.axis_index("c")
        s = lax.axis_index("s")
        r0 = s * ROWS_TILE
        crow = c * NP
        kchunk0 = s * NCHUNK
        si = 0
        bi = 0
        oi = 0

        for tspec, outs_cfg in rounds:
            y_ref = (tables[tspec[1]] if tspec[0] == "in"
                     else out_refs[tspec[1]])

            pltpu.sync_copy(y_ref.at[pl.ds(crow + r0, ROWS_TILE)],
                            acc.at[pl.ds(r0, ROWS_TILE)])
            plsc.subcore_barrier()

            idx_h = {}
            gat_h = {}
            scat_h = {}

            def issue_idx(k):
                idx_h[k] = pltpu.async_copy(
                    epack_ref.at[c * NCB + kchunk0 + k], ebuf.at[k % 4],
                    isems[k % 3])

            def issue_gather(k):
                gat_h[k] = pltpu.async_copy(
                    y_ref.at[ebuf.at[k % 4, 0]], rowbuf.at[k % 2],
                    gsems[k % 2])

            def issue_scat(k):
                scat_h[k] = pltpu.async_copy(
                    rowbuf.at[k % 2], acc.at[ebuf.at[k % 4, 1]],
                    ssems[k % 2], add=True)

            issue_idx(0)
            issue_idx(1)
            issue_idx(2)
            idx_h.pop(0).wait()
            issue_gather(0)
            for k in range(1, NCHUNK):
                idx_h.pop(k).wait()
                if k >= 2:
                    scat_h.pop(k - 2).wait()
                issue_gather(k)
                if k + 2 < NCHUNK:
                    issue_idx(k + 2)
                gat_h.pop(k - 1).wait()
                issue_scat(k - 1)
            gat_h.pop(NCHUNK - 1).wait()
            issue_scat(NCHUNK - 1)
            for k in sorted(scat_h):
                scat_h.pop(k).wait()
            plsc.subcore_barrier()

            for do_scale, do_bias in outs_cfg:
                dst_full = out_refs[oi].at[pl.ds(crow + r0, ROWS_TILE)]
                if not do_scale and not do_bias:
                    pltpu.sync_copy(acc.at[pl.ds(r0, ROWS_TILE)], dst_full)
                    oi += 1
                    continue
                if do_scale:
                    pltpu.sync_copy(
                        scale_refs[si].at[pl.ds(s * STILE, STILE)], sbuf)
                    si += 1
                if do_bias:
                    pltpu.sync_copy(bias_refs[bi].at[pl.ds(c * DH, DH)],
                                    bbuf)
                    bi += 1

                def one_row(wb, r, svb, _sc=do_scale, _bs=do_bias):
                    for g in range(DH // 16):
                        v = wb[r, pl.ds(g * 16, 16)]
                        if _sc:
                            v = v * svb
                        if _bs:
                            v = v + bbuf[pl.ds(g * 16, 16)]
                        wb[r, pl.ds(g * 16, 16)] = v

                out_h = {}
                for k, sz in enumerate(WCHUNKS):
                    b = k % 2
                    if k >= 2:
                        out_h.pop(k - 2).wait()
                    wb = wbuf.at[b]
                    pltpu.async_copy(acc.at[pl.ds(r0 + k * WCW, sz)],
                                     wb.at[pl.ds(0, sz)], wisem).wait()

                    def gbody(rg, carry, _k=k, _wb=wb, _sc=do_scale):
                        if _sc:
                            sv16 = sbuf[pl.ds(_k * WCW + rg * 8, 16)]
                        for j in range(8):
                            svb = (lax.broadcast(sv16[j], (16,))
                                   if _sc else None)
                            one_row(_wb, rg * 8 + j, svb)
                        return carry
                    lax.fori_loop(0, sz // 8, gbody, 0)
                    out_h[k] = pltpu.async_copy(
                        wb.at[pl.ds(0, sz)],
                        out_refs[oi].at[pl.ds(crow + r0 + k * WCW, sz)],
                        wosems[b])
                for k in sorted(out_h):
                    out_h.pop(k).wait()
                oi += 1

    out_type = [jax.ShapeDtypeStruct((2 * NP, DH), _f32)] * n_out
    scratch = [
        pltpu.VMEM((4, 2, CHUNK), jnp.int32),
        pltpu.VMEM((2, CHUNK, DH), _f32),
        pltpu.VMEM((2, WCW, DH), _f32),
        pltpu.VMEM((STILE,), _f32),
        pltpu.VMEM((DH,), _f32),
        pltpu.VMEM_SHARED((NP, DH), _f32),
    ] + [pltpu.SemaphoreType.DMA] * 10
    return pl.kernel(body, out_type=out_type, mesh=_mesh,
                     scratch_types=scratch, compiler_params=_sc_params)


_prop_plain_scaled = _make_prop(((("in", 0), ((False, False), (True, False))),),
                                n_tables=1)
_prop_plain = _make_prop(((("in", 0), ((False, False),)),), n_tables=1)
_prop_scaled = _make_prop(((("in", 0), ((True, False),)),), n_tables=1)
_prop_scaled_bias = _make_prop(((("in", 0), ((True, True),)),), n_tables=1)


_BMN = 1264
_GB = NP // _BMN


def _deg_reduce_body(degp_ref, d_ref, d2_ref, dinv_ref):
    deg = jnp.sum(degp_ref[...], axis=0, keepdims=True) + 1.0
    dv = lax.rsqrt(deg)
    d_ref[...] = dv
    d2_ref[...] = dv * dv
    dinv_ref[...] = deg * dv


def _tc_deg_reduce(degp):
    sds = jax.ShapeDtypeStruct((1, DEG_N), _f32)
    return pl.pallas_call(
        _deg_reduce_body,
        out_shape=[sds, sds, sds],
    )(degp)


def _scale_body(x_ref, d_ref, u0_ref):
    u0_ref[...] = x_ref[...] * d_ref[...]


def _tc_scale(x_p, d_np):
    return pl.pallas_call(
        _scale_body,
        grid=(_GB, 2),
        in_specs=[
            pl.BlockSpec((_BMN, DH), lambda i, j: (i, j)),
            pl.BlockSpec((_BMN, 1), lambda i, j: (i, 0)),
        ],
        out_specs=pl.BlockSpec((_BMN, DH), lambda i, j: (j * _GB + i, 0)),
        out_shape=jax.ShapeDtypeStruct((2 * NP, DH), _f32),
    )(x_p, d_np)


_bf16 = jnp.bfloat16


def _bdot(a, b):
    return jnp.dot(a.astype(_bf16), b.astype(_bf16),
                   preferred_element_type=_f32)


def _l1_body(x_ref, y2l_ref, y2r_ref, t2l_ref, t2r_ref, d_ref, di_ref,
             w0_ref, w1_ref, w2_ref, b_ref, h_ref):
    dv = d_ref[...]
    di = di_ref[...]
    w1 = w1_ref[...]
    w2 = w2_ref[...]
    a0 = _bdot(x_ref[...], w0_ref[...])
    a1 = _bdot(di * y2l_ref[...], w1[:DH]) + _bdot(di * y2r_ref[...], w1[DH:])
    a2 = _bdot(dv * t2l_ref[...], w2[:DH]) + _bdot(dv * t2r_ref[...], w2[DH:])
    h = jnp.concatenate([a0, a1, a2], axis=1) + b_ref[...]
    h_ref[...] = jnp.maximum(h, 0.0).astype(_bf16)


def _tc_layer1(x_p, y2c, t2c, d_np, dinv_np, w0, w1, w2, b):
    full = lambda r, c: pl.BlockSpec((r, c), lambda i: (0, 0))
    left = pl.BlockSpec((_BMN, DH), lambda i: (i, 0))
    right = pl.BlockSpec((_BMN, DH), lambda i: (i + _GB, 0))
    col = pl.BlockSpec((_BMN, 1), lambda i: (i, 0))
    return pl.pallas_call(
        _l1_body,
        grid=(_GB,),
        in_specs=[pl.BlockSpec((_BMN, D), lambda i: (i, 0)),
                  left, right, left, right, col, col,
                  full(D, D), full(D, D), full(D, D), full(1, 3 * D)],
        out_specs=pl.BlockSpec((_BMN, 3 * D), lambda i: (i, 0)),
        out_shape=jax.ShapeDtypeStruct((NP, 3 * D), _bf16),
    )(x_p, y2c, y2c, t2c, t2c, d_np, dinv_np, w0, w1, w2, b)


def _l2a_body(h_ref, d_ref, w1_ref, g1_ref):
    g1_ref[...] = d_ref[...] * jnp.dot(h_ref[...], w1_ref[...].astype(_bf16),
                                       preferred_element_type=_f32)


def _tc_layer2a(h_p, d_np, w1):
    return pl.pallas_call(
        _l2a_body,
        grid=(_GB, 2),
        in_specs=[pl.BlockSpec((_BMN, 3 * D), lambda i, j: (i, 0)),
                  pl.BlockSpec((_BMN, 1), lambda i, j: (i, 0)),
                  pl.BlockSpec((3 * D, DH), lambda i, j: (0, j))],
        out_specs=pl.BlockSpec((_BMN, DH), lambda i, j: (j * _GB + i, 0)),
        out_shape=jax.ShapeDtypeStruct((2 * NP, DH), _f32),
    )(h_p, d_np, w1)


def _l2b_body(h_ref, d_ref, w0_ref, w2_ref, b_ref, g0_ref, g2_ref):
    h = h_ref[...]
    g0_ref[...] = (jnp.dot(h, w0_ref[...].astype(_bf16),
                           preferred_element_type=_f32) + b_ref[...])
    g2_ref[...] = d_ref[...] * jnp.dot(h, w2_ref[...].astype(_bf16),
                                       preferred_element_type=_f32)


def _tc_layer2b(h_p, d_np, w0, w2, b):
    wspec = pl.BlockSpec((3 * D, DH), lambda i, j: (0, j))
    return pl.pallas_call(
        _l2b_body,
        grid=(_GB, 2),
        in_specs=[pl.BlockSpec((_BMN, 3 * D), lambda i, j: (i, 0)),
                  pl.BlockSpec((_BMN, 1), lambda i, j: (i, 0)),
                  wspec, wspec,
                  pl.BlockSpec((1, DH), lambda i, j: (0, j))],
        out_specs=[pl.BlockSpec((_BMN, DH), lambda i, j: (i, j)),
                   pl.BlockSpec((_BMN, DH), lambda i, j: (j * _GB + i, 0))],
        out_shape=[jax.ShapeDtypeStruct((NP, D), _f32),
                   jax.ShapeDtypeStruct((2 * NP, DH), _f32)],
    )(h_p, d_np, w0, w2, b)


def _scale_vec(v):
    v = v.reshape(NS, ROWS_TILE)
    return jnp.pad(v, ((0, 0), (0, STILE - ROWS_TILE))).reshape(-1)


def kernel(x, edge_index, W1_0, W1_1, W1_2, b1, W2_0, W2_1, W2_2, b2):
    src = edge_index[0]
    dst = edge_index[1]
    ar = jnp.arange(E_PAD - E, dtype=jnp.int32)
    ar_deg = jnp.arange(E_PAD_DEG - E, dtype=jnp.int32)
    src_p = jnp.concatenate([src, ar % 64])
    dst_p = jnp.concatenate([dst, N + (ar_deg % 16)])
    src2 = jnp.concatenate([src_p, src_p + NP]).reshape(2 * NCB, CHUNK)
    dst2 = jnp.concatenate([dst_p[:E_PAD]] * 2).reshape(2 * NCB, CHUNK)
    epack = jnp.stack([src2, dst2], axis=1)

    (degp,) = _deg_kernel(dst_p)
    drow, d2row, dinvrow = _tc_deg_reduce(degp)
    d_np = drow.reshape(NP, 1)
    dinv_np = dinvrow.reshape(NP, 1)
    d_t = _scale_vec(drow)
    d2_t = _scale_vec(d2row)

    x_p = jnp.pad(x, ((0, NP - N), (0, 0)))
    u0c = _tc_scale(x_p, d_np)

    (y2c,) = _prop_scaled(u0c, epack, d2_t)
    (t2c,) = _prop_plain(y2c, epack)

    h_p = _tc_layer1(x_p, y2c, t2c, d_np, dinv_np,
                     W1_0, W1_1, W1_2, b1.reshape(1, 3 * D))
    G1c = _tc_layer2a(h_p, d_np, W2_1)
    (q1c,) = _prop_scaled_bias(G1c, epack, d_t, b2[D:2 * D])
    g0_p, G2c = _tc_layer2b(h_p, d_np, W2_0, W2_2, b2[:D].reshape(1, D))
    (y4c,) = _prop_scaled(G2c, epack, d2_t)
    (q2c,) = _prop_scaled_bias(y4c, epack, d_t, b2[2 * D:])

    return jnp.concatenate([g0_p[:N], q1c[:N], q1c[NP:NP + N],
                            q2c[:N], q2c[NP:NP + N]], axis=1)


# speedup vs baseline: 20.3824x; 1.0693x over previous
import functools

import jax
import jax.numpy as jnp
from jax import lax
from jax.experimental import pallas as pl
from jax.experimental.pallas import tpu as pltpu
from jax.experimental.pallas import tpu_sc as plsc

N = 10000
E = 160000
D = 256
DH = 128
NC = 2
NS = 16
CHUNK = 128
NCHUNK = 79
EP_TILE = NCHUNK * CHUNK
E_PAD = NS * EP_TILE
NCB = E_PAD // CHUNK
NP = 10112
ROWS_TILE = NP // NS
DEG_TILE = 5120
E_PAD_DEG = NC * NS * DEG_TILE
DEG_N = NP
STILE = 640
WCW = 48
WCHUNKS = (48,) * 13 + (8,)

_f32 = jnp.float32
_mesh = plsc.VectorSubcoreMesh(
    core_axis_name="c", subcore_axis_name="s", num_cores=NC, num_subcores=NS)
_sc_params = pltpu.CompilerParams(needs_layout_passes=False)


def _deg_body(dst_ref, out_ref, dbuf, dacc):
    c = lax.axis_index("c")
    s = lax.axis_index("s")
    wid = c * NS + s
    zeros16 = jnp.zeros((16,), _f32)
    ones16 = jnp.ones((16,), _f32)

    def zbody(j, carry):
        dacc[pl.ds(j * 16, 16)] = zeros16
        return carry
    lax.fori_loop(0, DEG_N // 16, zbody, 0)

    pltpu.sync_copy(dst_ref.at[pl.ds(wid * DEG_TILE, DEG_TILE)], dbuf)

    def ebody(e, carry):
        idx = dbuf[pl.ds(e * 16, 16)]
        plsc.addupdate_scatter(dacc, [idx], ones16)
        return carry
    lax.fori_loop(0, DEG_TILE // 16, ebody, 0)

    pltpu.sync_copy(dacc.at[pl.ds(0, DEG_N)], out_ref.at[wid])


_deg_kernel = pl.kernel(
    _deg_body,
    out_type=[jax.ShapeDtypeStruct((NC * NS, DEG_N), _f32)],
    mesh=_mesh,
    scratch_types=[
        pltpu.VMEM((DEG_TILE,), jnp.int32),
        pltpu.VMEM((DEG_N,), _f32),
    ],
    compiler_params=_sc_params,
)


def _make_prop(rounds, n_tables):
    all_cfg = [c for _, outs in rounds for c in outs]
    n_out = len(all_cfg)
    n_scale = sum(1 for sc, _ in all_cfg if sc)
    n_bias = sum(1 for _, b in all_cfg if b)

    def body(*refs):
        pos = 0
        tables = refs[pos:pos + n_tables]; pos += n_tables
        epack_ref = refs[pos]; pos += 1
        scale_refs = refs[pos:pos + n_scale]; pos += n_scale
        bias_refs = refs[pos:pos + n_bias]; pos += n_bias
        out_refs = refs[pos:pos + n_out]; pos += n_out
        (ebuf, rowbuf, wbuf, sbuf, bbuf, acc,
         isem0, isem1, isem2, gsem0, gsem1, ssem0, ssem1,
         wisem, wosem0, wosem1) = refs[pos:]
        isems = (isem0, isem1, isem2)
        gsems = (gsem0, gsem1)
        ssems = (ssem0, ssem1)
        wosems = (wosem0, wosem1)

        c = lax.axis_index("c")
        s = lax.axis_index("s")
        r0 = s * ROWS_TILE
        crow = c * NP
        kchunk0 = s * NCHUNK
        si = 0
        bi = 0
        oi = 0

        for tspec, outs_cfg in rounds:
            y_ref = (tables[tspec[1]] if tspec[0] == "in"
                     else out_refs[tspec[1]])

            pltpu.sync_copy(y_ref.at[pl.ds(crow + r0, ROWS_TILE)],
                            acc.at[pl.ds(r0, ROWS_TILE)])
            plsc.subcore_barrier()

            idx_h = {}
            gat_h = {}
            scat_h = {}

            def issue_idx(k):
                idx_h[k] = pltpu.async_copy(
                    epack_ref.at[c * NCB + kchunk0 + k], ebuf.at[k % 4],
                    isems[k % 3])

            def issue_gather(k):
                gat_h[k] = pltpu.async_copy(
                    y_ref.at[ebuf.at[k % 4, 0]], rowbuf.at[k % 2],
                    gsems[k % 2])

            def issue_scat(k):
                scat_h[k] = pltpu.async_copy(
                    rowbuf.at[k % 2], acc.at[ebuf.at[k % 4, 1]],
                    ssems[k % 2], add=True)

            issue_idx(0)
            issue_idx(1)
            issue_idx(2)
            idx_h.pop(0).wait()
            issue_gather(0)
            for k in range(1, NCHUNK):
                idx_h.pop(k).wait()
                if k >= 2:
                    scat_h.pop(k - 2).wait()
                issue_gather(k)
                if k + 2 < NCHUNK:
                    issue_idx(k + 2)
                gat_h.pop(k - 1).wait()
                issue_scat(k - 1)
            gat_h.pop(NCHUNK - 1).wait()
            issue_scat(NCHUNK - 1)
            for k in sorted(scat_h):
                scat_h.pop(k).wait()
            plsc.subcore_barrier()

            for do_scale, do_bias in outs_cfg:
                dst_full = out_refs[oi].at[pl.ds(crow + r0, ROWS_TILE)]
                if not do_scale and not do_bias:
                    pltpu.sync_copy(acc.at[pl.ds(r0, ROWS_TILE)], dst_full)
                    oi += 1
                    continue
                if do_scale:
                    pltpu.sync_copy(
                        scale_refs[si].at[pl.ds(s * STILE, STILE)], sbuf)
                    si += 1
                if do_bias:
                    pltpu.sync_copy(bias_refs[bi].at[pl.ds(c * DH, DH)],
                                    bbuf)
                    bi += 1

                def one_row(wb, r, svb, _sc=do_scale, _bs=do_bias):
                    for g in range(DH // 16):
                        v = wb[r, pl.ds(g * 16, 16)]
                        if _sc:
                            v = v * svb
                        if _bs:
                            v = v + bbuf[pl.ds(g * 16, 16)]
                        wb[r, pl.ds(g * 16, 16)] = v

                out_h = {}
                for k, sz in enumerate(WCHUNKS):
                    b = k % 2
                    if k >= 2:
                        out_h.pop(k - 2).wait()
                    wb = wbuf.at[b]
                    pltpu.async_copy(acc.at[pl.ds(r0 + k * WCW, sz)],
                                     wb.at[pl.ds(0, sz)], wisem).wait()

                    def gbody(rg, carry, _k=k, _wb=wb, _sc=do_scale):
                        if _sc:
                            sv16 = sbuf[pl.ds(_k * WCW + rg * 8, 16)]
                        for j in range(8):
                            svb = (lax.broadcast(sv16[j], (16,))
                                   if _sc else None)
                            one_row(_wb, rg * 8 + j, svb)
                        return carry
                    lax.fori_loop(0, sz // 8, gbody, 0)
                    out_h[k] = pltpu.async_copy(
                        wb.at[pl.ds(0, sz)],
                        out_refs[oi].at[pl.ds(crow + r0 + k * WCW, sz)],
                        wosems[b])
                for k in sorted(out_h):
                    out_h.pop(k).wait()
                oi += 1

    out_type = [jax.ShapeDtypeStruct((2 * NP, DH), _f32)] * n_out
    scratch = [
        pltpu.VMEM((4, 2, CHUNK), jnp.int32),
        pltpu.VMEM((2, CHUNK, DH), _f32),
        pltpu.VMEM((2, WCW, DH), _f32),
        pltpu.VMEM((STILE,), _f32),
        pltpu.VMEM((DH,), _f32),
        pltpu.VMEM_SHARED((NP, DH), _f32),
    ] + [pltpu.SemaphoreType.DMA] * 10
    return pl.kernel(body, out_type=out_type, mesh=_mesh,
                     scratch_types=scratch, compiler_params=_sc_params)


_prop_plain_scaled = _make_prop(((("in", 0), ((False, False), (True, False))),),
                                n_tables=1)
_prop_plain = _make_prop(((("in", 0), ((False, False),)),), n_tables=1)
_prop_scaled = _make_prop(((("in", 0), ((True, False),)),), n_tables=1)
_prop_scaled_bias = _make_prop(((("in", 0), ((True, True),)),), n_tables=1)


_BMN = 1264
_GB = NP // _BMN


def _deg_reduce_body(degp_ref, d_ref, d2_ref, dinv_ref):
    deg = jnp.sum(degp_ref[...], axis=0, keepdims=True) + 1.0
    dv = lax.rsqrt(deg)
    d_ref[...] = dv
    d2_ref[...] = dv * dv
    dinv_ref[...] = deg * dv


def _tc_deg_reduce(degp):
    sds = jax.ShapeDtypeStruct((1, DEG_N), _f32)
    return pl.pallas_call(
        _deg_reduce_body,
        out_shape=[sds, sds, sds],
    )(degp)


def _scale_body(x_ref, d_ref, u0_ref):
    u0_ref[...] = x_ref[...] * d_ref[...]


def _tc_scale(x_p, d_np):
    return pl.pallas_call(
        _scale_body,
        grid=(_GB, 2),
        in_specs=[
            pl.BlockSpec((_BMN, DH), lambda i, j: (i, j)),
            pl.BlockSpec((_BMN, 1), lambda i, j: (i, 0)),
        ],
        out_specs=pl.BlockSpec((_BMN, DH), lambda i, j: (j * _GB + i, 0)),
        out_shape=jax.ShapeDtypeStruct((2 * NP, DH), _f32),
    )(x_p, d_np)


_bf16 = jnp.bfloat16


def _bdot(a, b):
    return jnp.dot(a.astype(_bf16), b.astype(_bf16),
                   preferred_element_type=_f32)


def _l1_body(x_ref, y2l_ref, y2r_ref, t2l_ref, t2r_ref, d_ref, di_ref,
             w0_ref, w1_ref, w2_ref, b_ref, h_ref):
    dv = d_ref[...]
    di = di_ref[...]
    w1 = w1_ref[...]
    w2 = w2_ref[...]
    a0 = _bdot(x_ref[...], w0_ref[...])
    a1 = _bdot(di * y2l_ref[...], w1[:DH]) + _bdot(di * y2r_ref[...], w1[DH:])
    a2 = _bdot(dv * t2l_ref[...], w2[:DH]) + _bdot(dv * t2r_ref[...], w2[DH:])
    h = jnp.concatenate([a0, a1, a2], axis=1) + b_ref[...]
    h_ref[...] = jnp.maximum(h, 0.0).astype(_bf16)


def _tc_layer1(x_p, y2c, t2c, d_np, dinv_np, w0, w1, w2, b):
    full = lambda r, c: pl.BlockSpec((r, c), lambda i: (0, 0))
    left = pl.BlockSpec((_BMN, DH), lambda i: (i, 0))
    right = pl.BlockSpec((_BMN, DH), lambda i: (i + _GB, 0))
    col = pl.BlockSpec((_BMN, 1), lambda i: (i, 0))
    return pl.pallas_call(
        _l1_body,
        grid=(_GB,),
        in_specs=[pl.BlockSpec((_BMN, D), lambda i: (i, 0)),
                  left, right, left, right, col, col,
                  full(D, D), full(D, D), full(D, D), full(1, 3 * D)],
        out_specs=pl.BlockSpec((_BMN, 3 * D), lambda i: (i, 0)),
        out_shape=jax.ShapeDtypeStruct((NP, 3 * D), _bf16),
    )(x_p, y2c, y2c, t2c, t2c, d_np, dinv_np, w0, w1, w2, b)


def _l2a_body(h_ref, d_ref, w1_ref, g1_ref):
    g1_ref[...] = d_ref[...] * jnp.dot(h_ref[...], w1_ref[...].astype(_bf16),
                                       preferred_element_type=_f32)


def _tc_layer2a(h_p, d_np, w1):
    return pl.pallas_call(
        _l2a_body,
        grid=(_GB, 2),
        in_specs=[pl.BlockSpec((_BMN, 3 * D), lambda i, j: (i, 0)),
                  pl.BlockSpec((_BMN, 1), lambda i, j: (i, 0)),
                  pl.BlockSpec((3 * D, DH), lambda i, j: (0, j))],
        out_specs=pl.BlockSpec((_BMN, DH), lambda i, j: (j * _GB + i, 0)),
        out_shape=jax.ShapeDtypeStruct((2 * NP, DH), _f32),
    )(h_p, d_np, w1)


def _l2b_body(h_ref, d_ref, w0_ref, w2_ref, b_ref, g0_ref, g2_ref):
    h = h_ref[...]
    g0_ref[...] = (jnp.dot(h, w0_ref[...].astype(_bf16),
                           preferred_element_type=_f32) + b_ref[...])
    g2_ref[...] = d_ref[...] * jnp.dot(h, w2_ref[...].astype(_bf16),
                                       preferred_element_type=_f32)


def _tc_layer2b(h_p, d_np, w0, w2, b):
    wspec = pl.BlockSpec((3 * D, DH), lambda i, j: (0, j))
    return pl.pallas_call(
        _l2b_body,
        grid=(_GB, 2),
        in_specs=[pl.BlockSpec((_BMN, 3 * D), lambda i, j: (i, 0)),
                  pl.BlockSpec((_BMN, 1), lambda i, j: (i, 0)),
                  wspec, wspec,
                  pl.BlockSpec((1, DH), lambda i, j: (0, j))],
        out_specs=[pl.BlockSpec((_BMN, DH), lambda i, j: (i, j)),
                   pl.BlockSpec((_BMN, DH), lambda i, j: (j * _GB + i, 0))],
        out_shape=[jax.ShapeDtypeStruct((NP, D), _f32),
                   jax.ShapeDtypeStruct((2 * NP, DH), _f32)],
    )(h_p, d_np, w0, w2, b)


def _asm_body(g0_ref, q1l_ref, q1r_ref, q2l_ref, q2r_ref, d_ref,
              b1_ref, b2_ref, out_ref):
    dv = d_ref[...]
    bq1 = b1_ref[...]
    bq2 = b2_ref[...]
    out_ref[...] = jnp.concatenate(
        [g0_ref[...],
         dv * q1l_ref[0] + bq1[:, :DH],
         dv * q1r_ref[0] + bq1[:, DH:],
         dv * q2l_ref[0] + bq2[:, :DH],
         dv * q2r_ref[0] + bq2[:, DH:]], axis=1)


def _tc_assemble(g0_p, q1r3, q2r3, d_np, bq1, bq2):
    bm = 2000
    half = lambda j: pl.BlockSpec((1, bm, DH), lambda i, _j=j: (_j, i, 0))
    bias = pl.BlockSpec((1, D), lambda i: (0, 0))
    return pl.pallas_call(
        _asm_body,
        grid=(N // bm,),
        in_specs=[pl.BlockSpec((bm, D), lambda i: (i, 0)),
                  half(0), half(1), half(0), half(1),
                  pl.BlockSpec((bm, 1), lambda i: (i, 0)),
                  bias, bias],
        out_specs=pl.BlockSpec((bm, 3 * D), lambda i: (i, 0)),
        out_shape=jax.ShapeDtypeStruct((N, 3 * D), _f32),
    )(g0_p, q1r3, q1r3, q2r3, q2r3, d_np, bq1, bq2)


def _scale_vec(v):
    v = v.reshape(NS, ROWS_TILE)
    return jnp.pad(v, ((0, 0), (0, STILE - ROWS_TILE))).reshape(-1)


def kernel(x, edge_index, W1_0, W1_1, W1_2, b1, W2_0, W2_1, W2_2, b2):
    src = edge_index[0]
    dst = edge_index[1]
    ar = jnp.arange(E_PAD - E, dtype=jnp.int32)
    ar_deg = jnp.arange(E_PAD_DEG - E, dtype=jnp.int32)
    src_p = jnp.concatenate([src, ar % 64])
    dst_p = jnp.concatenate([dst, N + (ar_deg % 16)])
    src2 = jnp.concatenate([src_p, src_p + NP]).reshape(2 * NCB, CHUNK)
    dst2 = jnp.concatenate([dst_p[:E_PAD]] * 2).reshape(2 * NCB, CHUNK)
    epack = jnp.stack([src2, dst2], axis=1)

    (degp,) = _deg_kernel(dst_p)
    drow, d2row, dinvrow = _tc_deg_reduce(degp)
    d_np = drow.reshape(NP, 1)
    dinv_np = dinvrow.reshape(NP, 1)
    d_t = _scale_vec(drow)
    d2_t = _scale_vec(d2row)

    x_p = jnp.pad(x, ((0, NP - N), (0, 0)))
    u0c = _tc_scale(x_p, d_np)

    (y2c,) = _prop_scaled(u0c, epack, d2_t)
    (t2c,) = _prop_plain(y2c, epack)

    h_p = _tc_layer1(x_p, y2c, t2c, d_np, dinv_np,
                     W1_0, W1_1, W1_2, b1.reshape(1, 3 * D))
    G1c = _tc_layer2a(h_p, d_np, W2_1)
    (q1r,) = _prop_plain(G1c, epack)
    g0_p, G2c = _tc_layer2b(h_p, d_np, W2_0, W2_2, b2[:D].reshape(1, D))
    (y4c,) = _prop_scaled(G2c, epack, d2_t)
    (q2r,) = _prop_plain(y4c, epack)

    return _tc_assemble(g0_p, q1r.reshape(2, NP, DH), q2r.reshape(2, NP, DH),
                        d_np, b2[D:2 * D].reshape(1, D),
                        b2[2 * D:].reshape(1, D))
